# baseline (speedup 1.0000x reference)
import jax
import jax.numpy as jnp
from jax import lax
from jax.experimental import pallas as pl
from jax.experimental.pallas import tpu as pltpu
from jax.experimental.pallas import tpu_sc as plsc

N = 10000
E = 320000
D = 128
H = 16
DE = 16

NC = 2
NS = 16
NW = NC * NS
EPW = E // NW
C = 80
NCH = EPW // C
NG = C // 16

_mesh = plsc.VectorSubcoreMesh(
    core_axis_name="c", subcore_axis_name="s", num_cores=NC, num_subcores=NS)

f32 = jnp.float32


def _edge_emb_body(ea_ref, we_ref, out_ref):
    out_ref[0] = jnp.dot(ea_ref[...], we_ref[0], preferred_element_type=f32)


def _edge_emb(edge_attr, we3):
    EB = 4000
    return pl.pallas_call(
        _edge_emb_body,
        grid=(3, E // EB),
        in_specs=[
            pl.BlockSpec((EB, DE), lambda l, i: (i, 0)),
            pl.BlockSpec((1, DE, H), lambda l, i: (l, 0, 0)),
        ],
        out_specs=pl.BlockSpec((1, EB, H), lambda l, i: (l, i, 0)),
        out_shape=jax.ShapeDtypeStruct((3, E, H), f32),
    )(edge_attr, we3)


def _proj_body(x_ref, wl_ref, bl_ref, wr_ref, br_ref, xl_ref, xr_ref):
    xv = x_ref[...]
    xl_ref[...] = jnp.dot(xv, wl_ref[...], preferred_element_type=f32) + bl_ref[0]
    xr_ref[...] = jnp.dot(xv, wr_ref[...], preferred_element_type=f32) + br_ref[0]


def _proj(x, wl, bl, wr, br):
    NB = 2000
    din = x.shape[1]
    return pl.pallas_call(
        _proj_body,
        grid=(N // NB,),
        in_specs=[
            pl.BlockSpec((NB, din), lambda i: (i, 0)),
            pl.BlockSpec((din, H), lambda i: (0, 0)),
            pl.BlockSpec((1, H), lambda i: (0, 0)),
            pl.BlockSpec((din, H), lambda i: (0, 0)),
            pl.BlockSpec((1, H), lambda i: (0, 0)),
        ],
        out_specs=[
            pl.BlockSpec((NB, H), lambda i: (i, 0)),
            pl.BlockSpec((NB, H), lambda i: (i, 0)),
        ],
        out_shape=[
            jax.ShapeDtypeStruct((N, H), f32),
            jax.ShapeDtypeStruct((N, H), f32),
        ],
    )(x, wl, bl, wr, br)


def _softmax_h(a0, a1, d0, d1, s0, s1, c0, c1, bias):
    sh0 = s0 / jnp.maximum(c0, 1.0)
    sh1 = s1 / jnp.maximum(c1, 1.0)
    sm = jnp.maximum(sh0, sh1)
    w0 = jnp.exp(sh0 - sm)
    w1 = jnp.exp(sh1 - sm)
    den = d0 * w0 + d1 * w1 + 1e-16
    acc = a0 * w0 + a1 * w1
    return acc / den + bias


_node_specs = [
    pl.BlockSpec((2000, H), lambda i: (i, 0)),
    pl.BlockSpec((2000, H), lambda i: (i, 0)),
    pl.BlockSpec((2000, 1), lambda i: (i, 0)),
    pl.BlockSpec((2000, 1), lambda i: (i, 0)),
    pl.BlockSpec((2000, 1), lambda i: (i, 0)),
    pl.BlockSpec((2000, 1), lambda i: (i, 0)),
    pl.BlockSpec((2000, 1), lambda i: (i, 0)),
    pl.BlockSpec((2000, 1), lambda i: (i, 0)),
    pl.BlockSpec((1, H), lambda i: (0, 0)),
]


def _combine_proj_body(a0_ref, a1_ref, d0_ref, d1_ref, s0_ref, s1_ref,
                       c0_ref, c1_ref, bias_ref,
                       wl_ref, bl_ref, wr_ref, br_ref, xl_ref, xr_ref):
    h = _softmax_h(a0_ref[...], a1_ref[...], d0_ref[...], d1_ref[...],
                   s0_ref[...], s1_ref[...], c0_ref[...], c1_ref[...],
                   bias_ref[0])
    xl_ref[...] = jnp.dot(h, wl_ref[...], preferred_element_type=f32) + bl_ref[0]
    xr_ref[...] = jnp.dot(h, wr_ref[...], preferred_element_type=f32) + br_ref[0]


def _combine_proj(a0, a1, d0, d1, s0, s1, c0, c1, bias, wl, bl, wr, br):
    return pl.pallas_call(
        _combine_proj_body,
        grid=(5,),
        in_specs=_node_specs + [
            pl.BlockSpec((H, H), lambda i: (0, 0)),
            pl.BlockSpec((1, H), lambda i: (0, 0)),
            pl.BlockSpec((H, H), lambda i: (0, 0)),
            pl.BlockSpec((1, H), lambda i: (0, 0)),
        ],
        out_specs=[
            pl.BlockSpec((2000, H), lambda i: (i, 0)),
            pl.BlockSpec((2000, H), lambda i: (i, 0)),
        ],
        out_shape=[
            jax.ShapeDtypeStruct((N, H), f32),
            jax.ShapeDtypeStruct((N, H), f32),
        ],
    )(a0, a1, d0, d1, s0, s1, c0, c1, bias, wl, bl, wr, br)


def _final_body(a0_ref, a1_ref, d0_ref, d1_ref, s0_ref, s1_ref,
                c0_ref, c1_ref, bias_ref, wlin_ref, blin_ref, out_ref):
    i = pl.program_id(0)
    h = _softmax_h(a0_ref[...], a1_ref[...], d0_ref[...], d1_ref[...],
                   s0_ref[...], s1_ref[...], c0_ref[...], c1_ref[...],
                   bias_ref[0])
    part = jnp.sum(jnp.dot(h, wlin_ref[...], preferred_element_type=f32))

    @pl.when(i == 0)
    def _():
        out_ref[...] = jnp.zeros((1, 1), f32)

    out_ref[...] += jnp.reshape(part / N, (1, 1))

    @pl.when(i == pl.num_programs(0) - 1)
    def _():
        out_ref[...] += blin_ref[...]


def _final(a0, a1, d0, d1, s0, s1, c0, c1, bias, wlin, blin):
    return pl.pallas_call(
        _final_body,
        grid=(5,),
        in_specs=_node_specs + [
            pl.BlockSpec((H, 1), lambda i: (0, 0)),
            pl.BlockSpec((1, 1), lambda i: (0, 0)),
        ],
        out_specs=pl.BlockSpec((1, 1), lambda i: (0, 0)),
        out_shape=jax.ShapeDtypeStruct((1, 1), f32),
    )(a0, a1, d0, d1, s0, s1, c0, c1, bias, wlin, blin)


_sc_params = pltpu.CompilerParams(
    needs_layout_passes=False, use_tc_tiling_on_sc=False)


def _make_layer_sc(layer):
    with_cnt = layer == 0

    def body(src2_hbm, dst2_hbm, xl_hbm, xr_hbm, e3_hbm, att_hbm,
             zn_hbm, znh_hbm, *rest):
        if with_cnt:
            (ssum0_hbm, ssum1_hbm, cnt0_hbm, cnt1_hbm,
             acc0_hbm, acc1_hbm, den0_hbm, den1_hbm,
             srcv2, dstv2, xlv0, xlv1, xrv0, xrv1, ev0, ev1,
             avb, exb, shift_v, t0v, c0v, onesv, attv,
             ssum_sh, cnt_sh, acc_sh, den_sh,
             sgl0, sgl1, sgr0, sgr1, se0, se1, sss, ssc, srs0, srs1, sds) = rest
            cnt0in_hbm = cnt1in_hbm = None
        else:
            (cnt0in_hbm, cnt1in_hbm,
             ssum0_hbm, ssum1_hbm,
             acc0_hbm, acc1_hbm, den0_hbm, den1_hbm,
             srcv2, dstv2, xlv0, xlv1, xrv0, xrv1, ev0, ev1,
             avb, exb, shift_v, t0v, c0v, attv,
             ssum_sh, acc_sh, den_sh,
             sgl0, sgl1, sgr0, sgr1, se0, se1, sss, srs0, srs1, sds) = rest
            cnt_sh = onesv = ssc = None
        xlv = [xlv0, xlv1]
        xrv = [xrv0, xrv1]
        ev = [ev0, ev1]
        rowv = xrv
        sgl = [sgl0, sgl1]
        sgr = [sgr0, sgr1]
        se = [se0, se1]
        srs = [srs0, srs1]

        c = lax.axis_index("c")
        s = lax.axis_index("s")
        wid = s * NC + c
        base0 = wid * EPW

        pltpu.sync_copy(att_hbm, attv)
        pltpu.sync_copy(src2_hbm.at[wid], srcv2)
        pltpu.sync_copy(dst2_hbm.at[wid], dstv2)

        if with_cnt:
            ones16 = jnp.full((16,), 1.0, f32)
            for b in range(NG):
                onesv[pl.ds(16 * b, 16)] = ones16

        @pl.when(s == 0)
        def _():
            pltpu.sync_copy(zn_hbm, ssum_sh)
            pltpu.sync_copy(znh_hbm, acc_sh)
            pltpu.sync_copy(zn_hbm, den_sh)
            if with_cnt:
                pltpu.sync_copy(zn_hbm, cnt_sh)

        plsc.subcore_barrier()

        attvec = attv[...]
        attks = [attvec[k] for k in range(H)]
        iota16 = lax.iota(jnp.int32, 16)
        kvecs = [jnp.full((16,), k, jnp.int32) for k in range(H)]

        def issue_g1(j, slot):
            pltpu.async_copy(xl_hbm.at[srcv2.at[j]], xlv[slot], sgl[slot])
            pltpu.async_copy(xr_hbm.at[dstv2.at[j]], xrv[slot], sgr[slot])
            pltpu.async_copy(e3_hbm.at[layer, pl.ds(base0 + j * C, C)],
                             ev[slot], se[slot])

        def process1(j, slot):
            pltpu.make_async_copy(xl_hbm.at[srcv2.at[j]], xlv[slot], sgl[slot]).wait()
            pltpu.make_async_copy(xr_hbm.at[dstv2.at[j]], xrv[slot], sgr[slot]).wait()
            pltpu.make_async_copy(e3_hbm.at[layer, pl.ds(base0 + j * C, C)],
                                  ev[slot], se[slot]).wait()
            for b in range(NG):
                ivec = iota16 + b * 16
                acc = jnp.zeros((16,), f32)
                for k in range(H):
                    z = (plsc.load_gather(xlv[slot], [ivec, kvecs[k]])
                         + plsc.load_gather(xrv[slot], [ivec, kvecs[k]])
                         + plsc.load_gather(ev[slot], [ivec, kvecs[k]]))
                    m = jnp.maximum(z, 0.2 * z)
                    acc = acc + m * attks[k]
                avb[pl.ds(j * C + b * 16, 16)] = acc
            pltpu.async_copy(avb.at[pl.ds(j * C, C)], ssum_sh.at[dstv2.at[j]],
                             sss, add=True)
            if with_cnt:
                pltpu.async_copy(onesv, cnt_sh.at[dstv2.at[j]], ssc, add=True)

        issue_g1(0, 0)

        def pair1(t, carry):
            a = 2 * t
            issue_g1(a + 1, 1)
            process1(a, 0)
            issue_g1(a + 2, 0)
            process1(a + 1, 1)
            return carry

        lax.fori_loop(0, NCH // 2, pair1, 0)
        process1(NCH - 1, 0)

        def drain1(i, carry):
            pltpu.make_async_copy(avb.at[pl.ds(0, C)], ssum_sh.at[dstv2.at[0]],
                                  sss).wait()
            if with_cnt:
                pltpu.make_async_copy(onesv, cnt_sh.at[dstv2.at[0]], ssc).wait()
            return carry

        lax.fori_loop(0, NCH, drain1, 0)

        plsc.subcore_barrier()

        pltpu.sync_copy(ssum_sh, t0v)
        if with_cnt:
            pltpu.sync_copy(cnt_sh, c0v)
        else:
            @pl.when(c == 0)
            def _():
                pltpu.sync_copy(cnt0in_hbm, c0v)

            @pl.when(c == 1)
            def _():
                pltpu.sync_copy(cnt1in_hbm, c0v)

        def sbody(i, carry):
            sl = pl.ds(i * 16, 16)
            shift_v[sl] = t0v[sl] / jnp.maximum(c0v[sl], 1.0)
            return carry

        lax.fori_loop(0, N // 16, sbody, 0)

        @pl.when((s == 0) & (c == 0))
        def _():
            pltpu.sync_copy(ssum_sh, ssum0_hbm)
            if with_cnt:
                pltpu.sync_copy(cnt_sh, cnt0_hbm)

        @pl.when((s == 0) & (c == 1))
        def _():
            pltpu.sync_copy(ssum_sh, ssum1_hbm)
            if with_cnt:
                pltpu.sync_copy(cnt_sh, cnt1_hbm)

        def issue_g2(j, slot):
            pltpu.async_copy(xl_hbm.at[srcv2.at[j]], xlv[slot], sgl[slot])

        zero16 = jnp.zeros((16,), f32)
        for slot in range(2):
            for i in range(C):
                rowv[slot][i, :] = zero16
            pltpu.async_copy(rowv[slot], acc_sh.at[dstv2.at[0]], srs[slot],
                             add=True)

        issue_g2(0, 0)

        def process2(j, slot):
            pltpu.make_async_copy(xl_hbm.at[srcv2.at[j]], xlv[slot], sgl[slot]).wait()
            pltpu.make_async_copy(rowv[slot], acc_sh.at[dstv2.at[0]],
                                  srs[slot]).wait()
            for b in range(NG):
                sl = pl.ds(j * C + b * 16, 16)
                dvec = dstv2[j, pl.ds(b * 16, 16)]
                svec = plsc.load_gather(shift_v, [dvec])
                exvec = jnp.exp(avb[sl] - svec)
                exb[sl] = exvec
                for t in range(16):
                    i = b * 16 + t
                    rowv[slot][i, :] = xlv[slot][i, :] * exvec[t]
            pltpu.async_copy(exb.at[pl.ds(j * C, C)], den_sh.at[dstv2.at[j]],
                             sds, add=True)
            pltpu.async_copy(rowv[slot], acc_sh.at[dstv2.at[j]], srs[slot],
                             add=True)

        def pair2(t, carry):
            a = 2 * t
            issue_g2(a + 1, 1)
            process2(a, 0)
            issue_g2(a + 2, 0)
            process2(a + 1, 1)
            return carry

        lax.fori_loop(0, NCH // 2, pair2, 0)
        process2(NCH - 1, 0)

        def drain2(i, carry):
            pltpu.make_async_copy(exb.at[pl.ds(0, C)], den_sh.at[dstv2.at[0]],
                                  sds).wait()
            return carry

        lax.fori_loop(0, NCH, drain2, 0)
        pltpu.make_async_copy(rowv[0], acc_sh.at[dstv2.at[0]], srs[0]).wait()
        pltpu.make_async_copy(rowv[1], acc_sh.at[dstv2.at[0]], srs[1]).wait()

        plsc.subcore_barrier()

        @pl.when((s == 0) & (c == 0))
        def _():
            pltpu.sync_copy(acc_sh, acc0_hbm)
            pltpu.sync_copy(den_sh, den0_hbm)

        @pl.when((s == 0) & (c == 1))
        def _():
            pltpu.sync_copy(acc_sh, acc1_hbm)
            pltpu.sync_copy(den_sh, den1_hbm)

    out_type = [jax.ShapeDtypeStruct((N,), f32),
                jax.ShapeDtypeStruct((N,), f32)]
    if with_cnt:
        out_type += [jax.ShapeDtypeStruct((N,), f32),
                     jax.ShapeDtypeStruct((N,), f32)]
    out_type += [jax.ShapeDtypeStruct((N, H), f32),
                 jax.ShapeDtypeStruct((N, H), f32),
                 jax.ShapeDtypeStruct((N,), f32),
                 jax.ShapeDtypeStruct((N,), f32)]
    scratch = [
        pltpu.VMEM((NCH, C), jnp.int32),
        pltpu.VMEM((NCH, C), jnp.int32),
        pltpu.VMEM((C, H), f32),
        pltpu.VMEM((C, H), f32),
        pltpu.VMEM((C, H), f32),
        pltpu.VMEM((C, H), f32),
        pltpu.VMEM((C, H), f32),
        pltpu.VMEM((C, H), f32),
        pltpu.VMEM((EPW,), f32),
        pltpu.VMEM((EPW,), f32),
        pltpu.VMEM((N,), f32),
        pltpu.VMEM((N,), f32),
        pltpu.VMEM((N,), f32),
    ]
    if with_cnt:
        scratch.append(pltpu.VMEM((C,), f32))
    scratch.append(pltpu.VMEM((16,), f32))
    scratch.append(pltpu.VMEM_SHARED((N,), f32))
    if with_cnt:
        scratch.append(pltpu.VMEM_SHARED((N,), f32))
    scratch.append(pltpu.VMEM_SHARED((N, H), f32))
    scratch.append(pltpu.VMEM_SHARED((N,), f32))
    scratch += [pltpu.SemaphoreType.DMA] * (11 if with_cnt else 10)

    return pl.kernel(body, out_type=out_type, mesh=_mesh,
                     scratch_types=scratch, compiler_params=_sc_params)


_layer_sc = [_make_layer_sc(0), _make_layer_sc(1), _make_layer_sc(2)]


def kernel(x, edge_index, edge_attr, params, Wlin, blin):
    src = edge_index[0].astype(jnp.int32)
    dst = edge_index[1].astype(jnp.int32)
    src2 = src.reshape(NW, NCH, C)
    dst2 = dst.reshape(NW, NCH, C)

    we3 = jnp.stack([p[4] for p in params])
    e3 = _edge_emb(edge_attr, we3)

    zn = jnp.zeros((N,), f32)
    znh = jnp.zeros((N, H), f32)

    wl, bl, wr, br, _, att, bias = params[0]
    xl, xr = _proj(x, wl, bl.reshape(1, H), wr, br.reshape(1, H))

    out = None
    c0 = c1 = None
    for l in range(3):
        if l == 0:
            s0, s1, c0, c1, a0, a1, d0, d1 = _layer_sc[0](
                src2, dst2, xl, xr, e3, att, zn, znh)
        else:
            s0, s1, a0, a1, d0, d1 = _layer_sc[l](
                src2, dst2, xl, xr, e3, att, zn, znh, c0, c1)
        col = lambda v: v.reshape(N, 1)
        args = (a0, a1, col(d0), col(d1), col(s0), col(s1), col(c0), col(c1),
                bias.reshape(1, H))
        if l < 2:
            nwl, nbl, nwr, nbr, _, natt, nbias = params[l + 1]
            xl, xr = _combine_proj(*args, nwl, nbl.reshape(1, H),
                                   nwr, nbr.reshape(1, H))
            att = natt
            bias = nbias
        else:
            out = _final(*args, Wlin, blin.reshape(1, 1))
    return out


# speedup vs baseline: 1.2811x; 1.2811x over previous
import jax
import jax.numpy as jnp
from jax import lax
from jax.experimental import pallas as pl
from jax.experimental.pallas import tpu as pltpu
from jax.experimental.pallas import tpu_sc as plsc

N = 10000
E = 320000
D = 128
H = 16
DE = 16

NC = 2
NS = 16
NW = NC * NS
EPW = E // NW
C = 80
NCH = EPW // C
NG = C // 16

_mesh = plsc.VectorSubcoreMesh(
    core_axis_name="c", subcore_axis_name="s", num_cores=NC, num_subcores=NS)

f32 = jnp.float32


def _edge_emb_body(ea_ref, we_ref, o0_ref, o1_ref, o2_ref):
    ea = ea_ref[...]
    o0_ref[...] = jnp.dot(ea, we_ref[0], preferred_element_type=f32)
    o1_ref[...] = jnp.dot(ea, we_ref[1], preferred_element_type=f32)
    o2_ref[...] = jnp.dot(ea, we_ref[2], preferred_element_type=f32)


def _edge_emb(edge_attr, we3):
    EB = 4000
    return pl.pallas_call(
        _edge_emb_body,
        grid=(E // EB,),
        in_specs=[
            pl.BlockSpec((EB, DE), lambda i: (i, 0)),
            pl.BlockSpec((3, DE, H), lambda i: (0, 0, 0)),
        ],
        out_specs=[pl.BlockSpec((EB, H), lambda i: (i, 0))] * 3,
        out_shape=[jax.ShapeDtypeStruct((E, H), f32)] * 3,
    )(edge_attr, we3)


def _proj_body(x_ref, wl_ref, bl_ref, wr_ref, br_ref, xl_ref, xr_ref):
    xv = x_ref[...]
    xl_ref[...] = jnp.dot(xv, wl_ref[...], preferred_element_type=f32) + bl_ref[0]
    xr_ref[...] = jnp.dot(xv, wr_ref[...], preferred_element_type=f32) + br_ref[0]


def _proj(x, wl, bl, wr, br):
    NB = 2000
    din = x.shape[1]
    return pl.pallas_call(
        _proj_body,
        grid=(N // NB,),
        in_specs=[
            pl.BlockSpec((NB, din), lambda i: (i, 0)),
            pl.BlockSpec((din, H), lambda i: (0, 0)),
            pl.BlockSpec((1, H), lambda i: (0, 0)),
            pl.BlockSpec((din, H), lambda i: (0, 0)),
            pl.BlockSpec((1, H), lambda i: (0, 0)),
        ],
        out_specs=[
            pl.BlockSpec((NB, H), lambda i: (i, 0)),
            pl.BlockSpec((NB, H), lambda i: (i, 0)),
        ],
        out_shape=[
            jax.ShapeDtypeStruct((N, H), f32),
            jax.ShapeDtypeStruct((N, H), f32),
        ],
    )(x, wl, bl, wr, br)


def _softmax_h(a0, a1, d0, d1, s0, s1, c0, c1, bias):
    sh0 = s0 / jnp.maximum(c0, 1.0)
    sh1 = s1 / jnp.maximum(c1, 1.0)
    sm = jnp.maximum(sh0, sh1)
    w0 = jnp.exp(sh0 - sm)
    w1 = jnp.exp(sh1 - sm)
    den = d0 * w0 + d1 * w1 + 1e-16
    acc = a0 * w0 + a1 * w1
    return acc / den + bias


_node_specs = [
    pl.BlockSpec((2000, H), lambda i: (i, 0)),
    pl.BlockSpec((2000, H), lambda i: (i, 0)),
    pl.BlockSpec((2000, 1), lambda i: (i, 0)),
    pl.BlockSpec((2000, 1), lambda i: (i, 0)),
    pl.BlockSpec((2000, 1), lambda i: (i, 0)),
    pl.BlockSpec((2000, 1), lambda i: (i, 0)),
    pl.BlockSpec((2000, 1), lambda i: (i, 0)),
    pl.BlockSpec((2000, 1), lambda i: (i, 0)),
    pl.BlockSpec((1, H), lambda i: (0, 0)),
]


def _combine_proj_body(a0_ref, a1_ref, d0_ref, d1_ref, s0_ref, s1_ref,
                       c0_ref, c1_ref, bias_ref,
                       wl_ref, bl_ref, wr_ref, br_ref, xl_ref, xr_ref):
    h = _softmax_h(a0_ref[...], a1_ref[...], d0_ref[...], d1_ref[...],
                   s0_ref[...], s1_ref[...], c0_ref[...], c1_ref[...],
                   bias_ref[0])
    xl_ref[...] = jnp.dot(h, wl_ref[...], preferred_element_type=f32) + bl_ref[0]
    xr_ref[...] = jnp.dot(h, wr_ref[...], preferred_element_type=f32) + br_ref[0]


def _combine_proj(a0, a1, d0, d1, s0, s1, c0, c1, bias, wl, bl, wr, br):
    return pl.pallas_call(
        _combine_proj_body,
        grid=(5,),
        in_specs=_node_specs + [
            pl.BlockSpec((H, H), lambda i: (0, 0)),
            pl.BlockSpec((1, H), lambda i: (0, 0)),
            pl.BlockSpec((H, H), lambda i: (0, 0)),
            pl.BlockSpec((1, H), lambda i: (0, 0)),
        ],
        out_specs=[
            pl.BlockSpec((2000, H), lambda i: (i, 0)),
            pl.BlockSpec((2000, H), lambda i: (i, 0)),
        ],
        out_shape=[
            jax.ShapeDtypeStruct((N, H), f32),
            jax.ShapeDtypeStruct((N, H), f32),
        ],
    )(a0, a1, d0, d1, s0, s1, c0, c1, bias, wl, bl, wr, br)


def _final_body(a0_ref, a1_ref, d0_ref, d1_ref, s0_ref, s1_ref,
                c0_ref, c1_ref, bias_ref, wlin_ref, blin_ref, out_ref):
    i = pl.program_id(0)
    h = _softmax_h(a0_ref[...], a1_ref[...], d0_ref[...], d1_ref[...],
                   s0_ref[...], s1_ref[...], c0_ref[...], c1_ref[...],
                   bias_ref[0])
    part = jnp.sum(jnp.dot(h, wlin_ref[...], preferred_element_type=f32))

    @pl.when(i == 0)
    def _():
        out_ref[...] = jnp.zeros((1, 1), f32)

    out_ref[...] += jnp.reshape(part / N, (1, 1))

    @pl.when(i == pl.num_programs(0) - 1)
    def _():
        out_ref[...] += blin_ref[...]


def _final(a0, a1, d0, d1, s0, s1, c0, c1, bias, wlin, blin):
    return pl.pallas_call(
        _final_body,
        grid=(5,),
        in_specs=_node_specs + [
            pl.BlockSpec((H, 1), lambda i: (0, 0)),
            pl.BlockSpec((1, 1), lambda i: (0, 0)),
        ],
        out_specs=pl.BlockSpec((1, 1), lambda i: (0, 0)),
        out_shape=jax.ShapeDtypeStruct((1, 1), f32),
    )(a0, a1, d0, d1, s0, s1, c0, c1, bias, wlin, blin)


_sc_params = pltpu.CompilerParams(
    needs_layout_passes=False, use_tc_tiling_on_sc=False)


def _make_layer_sc():
    def body(src2_hbm, dst2_hbm, xl_hbm, xr_hbm, el_hbm, att_hbm,
             zn_hbm, znh_hbm, *rest):
        (ssum0_hbm, ssum1_hbm, cnt0_hbm, cnt1_hbm,
         acc0_hbm, acc1_hbm, den0_hbm, den1_hbm,
         srcv2, dstv2, xlv0, xlv1, xrv0, xrv1, ev0, ev1,
         avb, exb, shift_v, t0v, c0v, onesv, attv,
         ssum_sh, cnt_sh, acc_sh, den_sh,
         sgl0, sgl1, sgr0, sgr1, se0, se1, sss, ssc, srs0, srs1, sds) = rest
        xlv = [xlv0, xlv1]
        xrv = [xrv0, xrv1]
        ev = [ev0, ev1]
        rowv = xrv
        sgl = [sgl0, sgl1]
        sgr = [sgr0, sgr1]
        se = [se0, se1]
        srs = [srs0, srs1]

        c = lax.axis_index("c")
        s = lax.axis_index("s")
        wid = s * NC + c
        base0 = wid * EPW

        pltpu.sync_copy(att_hbm, attv)
        pltpu.sync_copy(src2_hbm.at[wid], srcv2)
        pltpu.sync_copy(dst2_hbm.at[wid], dstv2)

        ones16 = jnp.full((16,), 1.0, f32)
        for b in range(NG):
            onesv[pl.ds(16 * b, 16)] = ones16

        @pl.when(s == 0)
        def _():
            pltpu.sync_copy(zn_hbm, ssum_sh)
            pltpu.sync_copy(znh_hbm, acc_sh)
            pltpu.sync_copy(zn_hbm, den_sh)
            pltpu.sync_copy(zn_hbm, cnt_sh)

        plsc.subcore_barrier()

        attvec = attv[...]
        attks = [attvec[k] for k in range(H)]
        iota16 = lax.iota(jnp.int32, 16)
        kvecs = [jnp.full((16,), k, jnp.int32) for k in range(H)]

        def issue_g1(j, slot):
            pltpu.async_copy(xl_hbm.at[srcv2.at[j]], xlv[slot], sgl[slot])
            pltpu.async_copy(xr_hbm.at[dstv2.at[j]], xrv[slot], sgr[slot])
            pltpu.async_copy(el_hbm.at[pl.ds(base0 + j * C, C)],
                             ev[slot], se[slot])

        def process1(j, slot):
            pltpu.make_async_copy(xl_hbm.at[srcv2.at[j]], xlv[slot], sgl[slot]).wait()
            pltpu.make_async_copy(xr_hbm.at[dstv2.at[j]], xrv[slot], sgr[slot]).wait()
            pltpu.make_async_copy(el_hbm.at[pl.ds(base0 + j * C, C)],
                                  ev[slot], se[slot]).wait()
            for b in range(NG):
                ivec = iota16 + b * 16
                acc = jnp.zeros((16,), f32)
                for k in range(H):
                    z = (plsc.load_gather(xlv[slot], [ivec, kvecs[k]])
                         + plsc.load_gather(xrv[slot], [ivec, kvecs[k]])
                         + plsc.load_gather(ev[slot], [ivec, kvecs[k]]))
                    m = jnp.maximum(z, 0.2 * z)
                    acc = acc + m * attks[k]
                avb[pl.ds(j * C + b * 16, 16)] = acc
            pltpu.async_copy(avb.at[pl.ds(j * C, C)], ssum_sh.at[dstv2.at[j]],
                             sss, add=True)
            pltpu.async_copy(onesv, cnt_sh.at[dstv2.at[j]], ssc, add=True)

        issue_g1(0, 0)

        def pair1(t, carry):
            a = 2 * t
            issue_g1(a + 1, 1)
            process1(a, 0)
            issue_g1(a + 2, 0)
            process1(a + 1, 1)
            return carry

        lax.fori_loop(0, NCH // 2, pair1, 0)
        process1(NCH - 1, 0)

        def drain1(i, carry):
            pltpu.make_async_copy(avb.at[pl.ds(0, C)], ssum_sh.at[dstv2.at[0]],
                                  sss).wait()
            pltpu.make_async_copy(onesv, cnt_sh.at[dstv2.at[0]], ssc).wait()
            return carry

        lax.fori_loop(0, NCH, drain1, 0)

        plsc.subcore_barrier()

        pltpu.sync_copy(ssum_sh, t0v)
        pltpu.sync_copy(cnt_sh, c0v)

        def sbody(i, carry):
            sl = pl.ds(i * 16, 16)
            shift_v[sl] = t0v[sl] / jnp.maximum(c0v[sl], 1.0)
            return carry

        lax.fori_loop(0, N // 16, sbody, 0)

        @pl.when((s == 0) & (c == 0))
        def _():
            pltpu.sync_copy(ssum_sh, ssum0_hbm)
            pltpu.sync_copy(cnt_sh, cnt0_hbm)

        @pl.when((s == 0) & (c == 1))
        def _():
            pltpu.sync_copy(ssum_sh, ssum1_hbm)
            pltpu.sync_copy(cnt_sh, cnt1_hbm)

        def issue_g2(j, slot):
            pltpu.async_copy(xl_hbm.at[srcv2.at[j]], xlv[slot], sgl[slot])

        zero16 = jnp.zeros((16,), f32)
        for slot in range(2):
            for i in range(C):
                rowv[slot][i, :] = zero16
            pltpu.async_copy(rowv[slot], acc_sh.at[dstv2.at[0]], srs[slot],
                             add=True)

        issue_g2(0, 0)

        def process2(j, slot):
            pltpu.make_async_copy(xl_hbm.at[srcv2.at[j]], xlv[slot], sgl[slot]).wait()
            pltpu.make_async_copy(rowv[slot], acc_sh.at[dstv2.at[0]],
                                  srs[slot]).wait()
            for b in range(NG):
                sl = pl.ds(j * C + b * 16, 16)
                dvec = dstv2[j, pl.ds(b * 16, 16)]
                svec = plsc.load_gather(shift_v, [dvec])
                exvec = jnp.exp(avb[sl] - svec)
                exb[sl] = exvec
                for t in range(16):
                    i = b * 16 + t
                    rowv[slot][i, :] = xlv[slot][i, :] * exvec[t]
            pltpu.async_copy(exb.at[pl.ds(j * C, C)], den_sh.at[dstv2.at[j]],
                             sds, add=True)
            pltpu.async_copy(rowv[slot], acc_sh.at[dstv2.at[j]], srs[slot],
                             add=True)

        def pair2(t, carry):
            a = 2 * t
            issue_g2(a + 1, 1)
            process2(a, 0)
            issue_g2(a + 2, 0)
            process2(a + 1, 1)
            return carry

        lax.fori_loop(0, NCH // 2, pair2, 0)
        process2(NCH - 1, 0)

        def drain2(i, carry):
            pltpu.make_async_copy(exb.at[pl.ds(0, C)], den_sh.at[dstv2.at[0]],
                                  sds).wait()
            return carry

        lax.fori_loop(0, NCH, drain2, 0)
        pltpu.make_async_copy(rowv[0], acc_sh.at[dstv2.at[0]], srs[0]).wait()
        pltpu.make_async_copy(rowv[1], acc_sh.at[dstv2.at[0]], srs[1]).wait()

        plsc.subcore_barrier()

        @pl.when((s == 0) & (c == 0))
        def _():
            pltpu.sync_copy(acc_sh, acc0_hbm)
            pltpu.sync_copy(den_sh, den0_hbm)

        @pl.when((s == 0) & (c == 1))
        def _():
            pltpu.sync_copy(acc_sh, acc1_hbm)
            pltpu.sync_copy(den_sh, den1_hbm)

    out_type = [jax.ShapeDtypeStruct((N,), f32),
                jax.ShapeDtypeStruct((N,), f32),
                jax.ShapeDtypeStruct((N,), f32),
                jax.ShapeDtypeStruct((N,), f32),
                jax.ShapeDtypeStruct((N, H), f32),
                jax.ShapeDtypeStruct((N, H), f32),
                jax.ShapeDtypeStruct((N,), f32),
                jax.ShapeDtypeStruct((N,), f32)]
    scratch = [
        pltpu.VMEM((NCH, C), jnp.int32),
        pltpu.VMEM((NCH, C), jnp.int32),
        pltpu.VMEM((C, H), f32),
        pltpu.VMEM((C, H), f32),
        pltpu.VMEM((C, H), f32),
        pltpu.VMEM((C, H), f32),
        pltpu.VMEM((C, H), f32),
        pltpu.VMEM((C, H), f32),
        pltpu.VMEM((EPW,), f32),
        pltpu.VMEM((EPW,), f32),
        pltpu.VMEM((N,), f32),
        pltpu.VMEM((N,), f32),
        pltpu.VMEM((N,), f32),
        pltpu.VMEM((C,), f32),
        pltpu.VMEM((16,), f32),
        pltpu.VMEM_SHARED((N,), f32),
        pltpu.VMEM_SHARED((N,), f32),
        pltpu.VMEM_SHARED((N, H), f32),
        pltpu.VMEM_SHARED((N,), f32),
    ]
    scratch += [pltpu.SemaphoreType.DMA] * 11

    return pl.kernel(body, out_type=out_type, mesh=_mesh,
                     scratch_types=scratch, compiler_params=_sc_params)


_layer_sc = _make_layer_sc()


def kernel(x, edge_index, edge_attr, params, Wlin, blin):
    src = edge_index[0].astype(jnp.int32)
    dst = edge_index[1].astype(jnp.int32)
    src2 = src.reshape(NW, NCH, C)
    dst2 = dst.reshape(NW, NCH, C)

    we3 = jnp.stack([p[4] for p in params])
    els = _edge_emb(edge_attr, we3)

    zn = jnp.zeros((N,), f32)
    znh = jnp.zeros((N, H), f32)

    wl, bl, wr, br, _, att, bias = params[0]
    xl, xr = _proj(x, wl, bl.reshape(1, H), wr, br.reshape(1, H))

    out = None
    for l in range(3):
        s0, s1, c0, c1, a0, a1, d0, d1 = _layer_sc(
            src2, dst2, xl, xr, els[l], att, zn, znh)
        col = lambda v: v.reshape(N, 1)
        args = (a0, a1, col(d0), col(d1), col(s0), col(s1), col(c0), col(c1),
                bias.reshape(1, H))
        if l < 2:
            nwl, nbl, nwr, nbr, _, natt, nbias = params[l + 1]
            xl, xr = _combine_proj(*args, nwl, nbl.reshape(1, H),
                                   nwr, nbr.reshape(1, H))
            att = natt
            bias = nbias
        else:
            out = _final(*args, Wlin, blin.reshape(1, 1))
    return out


# speedup vs baseline: 1.2928x; 1.0092x over previous
import jax
import jax.numpy as jnp
from jax import lax
from jax.experimental import pallas as pl
from jax.experimental.pallas import tpu as pltpu
from jax.experimental.pallas import tpu_sc as plsc

N = 10000
E = 320000
D = 128
H = 16
DE = 16

NC = 2
NS = 16
NW = NC * NS
EPW = E // NW
C = 80
NCH = EPW // C
NG = C // 16

_mesh = plsc.VectorSubcoreMesh(
    core_axis_name="c", subcore_axis_name="s", num_cores=NC, num_subcores=NS)

f32 = jnp.float32


def _edge_emb_body(ea_ref, we_ref, o0_ref, o1_ref, o2_ref):
    ea = ea_ref[...]
    o0_ref[...] = jnp.dot(ea, we_ref[0], preferred_element_type=f32)
    o1_ref[...] = jnp.dot(ea, we_ref[1], preferred_element_type=f32)
    o2_ref[...] = jnp.dot(ea, we_ref[2], preferred_element_type=f32)


def _edge_emb(edge_attr, we3):
    EB = 4000
    return pl.pallas_call(
        _edge_emb_body,
        grid=(E // EB,),
        in_specs=[
            pl.BlockSpec((EB, DE), lambda i: (i, 0)),
            pl.BlockSpec((3, DE, H), lambda i: (0, 0, 0)),
        ],
        out_specs=[pl.BlockSpec((EB, H), lambda i: (i, 0))] * 3,
        out_shape=[jax.ShapeDtypeStruct((E, H), f32)] * 3,
    )(edge_attr, we3)


def _proj_body(x_ref, wl_ref, bl_ref, wr_ref, br_ref, xl_ref, xr_ref):
    xv = x_ref[...]
    xl_ref[...] = jnp.dot(xv, wl_ref[...], preferred_element_type=f32) + bl_ref[0]
    xr_ref[...] = jnp.dot(xv, wr_ref[...], preferred_element_type=f32) + br_ref[0]


def _proj(x, wl, bl, wr, br):
    NB = 2000
    din = x.shape[1]
    return pl.pallas_call(
        _proj_body,
        grid=(N // NB,),
        in_specs=[
            pl.BlockSpec((NB, din), lambda i: (i, 0)),
            pl.BlockSpec((din, H), lambda i: (0, 0)),
            pl.BlockSpec((1, H), lambda i: (0, 0)),
            pl.BlockSpec((din, H), lambda i: (0, 0)),
            pl.BlockSpec((1, H), lambda i: (0, 0)),
        ],
        out_specs=[
            pl.BlockSpec((NB, H), lambda i: (i, 0)),
            pl.BlockSpec((NB, H), lambda i: (i, 0)),
        ],
        out_shape=[
            jax.ShapeDtypeStruct((N, H), f32),
            jax.ShapeDtypeStruct((N, H), f32),
        ],
    )(x, wl, bl, wr, br)


def _softmax_h(a0, a1, d0, d1, s0, s1, c0, c1, bias):
    sh0 = s0 / jnp.maximum(c0, 1.0)
    sh1 = s1 / jnp.maximum(c1, 1.0)
    sm = jnp.maximum(sh0, sh1)
    w0 = jnp.exp(sh0 - sm)
    w1 = jnp.exp(sh1 - sm)
    den = d0 * w0 + d1 * w1 + 1e-16
    acc = a0 * w0 + a1 * w1
    return acc / den + bias


_node_specs = [
    pl.BlockSpec((2000, H), lambda i: (i, 0)),
    pl.BlockSpec((2000, H), lambda i: (i, 0)),
    pl.BlockSpec((2000, 1), lambda i: (i, 0)),
    pl.BlockSpec((2000, 1), lambda i: (i, 0)),
    pl.BlockSpec((2000, 1), lambda i: (i, 0)),
    pl.BlockSpec((2000, 1), lambda i: (i, 0)),
    pl.BlockSpec((2000, 1), lambda i: (i, 0)),
    pl.BlockSpec((2000, 1), lambda i: (i, 0)),
    pl.BlockSpec((1, H), lambda i: (0, 0)),
]


def _combine_proj_body(a0_ref, a1_ref, d0_ref, d1_ref, s0_ref, s1_ref,
                       c0_ref, c1_ref, bias_ref,
                       wl_ref, bl_ref, wr_ref, br_ref, xl_ref, xr_ref):
    h = _softmax_h(a0_ref[...], a1_ref[...], d0_ref[...], d1_ref[...],
                   s0_ref[...], s1_ref[...], c0_ref[...], c1_ref[...],
                   bias_ref[0])
    xl_ref[...] = jnp.dot(h, wl_ref[...], preferred_element_type=f32) + bl_ref[0]
    xr_ref[...] = jnp.dot(h, wr_ref[...], preferred_element_type=f32) + br_ref[0]


def _combine_proj(a0, a1, d0, d1, s0, s1, c0, c1, bias, wl, bl, wr, br):
    return pl.pallas_call(
        _combine_proj_body,
        grid=(5,),
        in_specs=_node_specs + [
            pl.BlockSpec((H, H), lambda i: (0, 0)),
            pl.BlockSpec((1, H), lambda i: (0, 0)),
            pl.BlockSpec((H, H), lambda i: (0, 0)),
            pl.BlockSpec((1, H), lambda i: (0, 0)),
        ],
        out_specs=[
            pl.BlockSpec((2000, H), lambda i: (i, 0)),
            pl.BlockSpec((2000, H), lambda i: (i, 0)),
        ],
        out_shape=[
            jax.ShapeDtypeStruct((N, H), f32),
            jax.ShapeDtypeStruct((N, H), f32),
        ],
    )(a0, a1, d0, d1, s0, s1, c0, c1, bias, wl, bl, wr, br)


def _final_body(a0_ref, a1_ref, d0_ref, d1_ref, s0_ref, s1_ref,
                c0_ref, c1_ref, bias_ref, wlin_ref, blin_ref, out_ref):
    i = pl.program_id(0)
    h = _softmax_h(a0_ref[...], a1_ref[...], d0_ref[...], d1_ref[...],
                   s0_ref[...], s1_ref[...], c0_ref[...], c1_ref[...],
                   bias_ref[0])
    part = jnp.sum(jnp.dot(h, wlin_ref[...], preferred_element_type=f32))

    @pl.when(i == 0)
    def _():
        out_ref[...] = jnp.zeros((1, 1), f32)

    out_ref[...] += jnp.reshape(part / N, (1, 1))

    @pl.when(i == pl.num_programs(0) - 1)
    def _():
        out_ref[...] += blin_ref[...]


def _final(a0, a1, d0, d1, s0, s1, c0, c1, bias, wlin, blin):
    return pl.pallas_call(
        _final_body,
        grid=(5,),
        in_specs=_node_specs + [
            pl.BlockSpec((H, 1), lambda i: (0, 0)),
            pl.BlockSpec((1, 1), lambda i: (0, 0)),
        ],
        out_specs=pl.BlockSpec((1, 1), lambda i: (0, 0)),
        out_shape=jax.ShapeDtypeStruct((1, 1), f32),
    )(a0, a1, d0, d1, s0, s1, c0, c1, bias, wlin, blin)


_sc_params = pltpu.CompilerParams(
    needs_layout_passes=False, use_tc_tiling_on_sc=False)


def _make_layer_sc():
    def body(src2_hbm, dst2_hbm, xl_hbm, xr_hbm, el_hbm, att_hbm,
             zn_hbm, znh_hbm, *rest):
        (ssum0_hbm, ssum1_hbm, cnt0_hbm, cnt1_hbm,
         acc0_hbm, acc1_hbm, den0_hbm, den1_hbm,
         srcv2, dstv2, xlv0, xlv1, xrv0, xrv1, ev0, ev1,
         avb, exb, shift_v, t0v, c0v, onesv, attv,
         ssum_sh, cnt_sh, acc_sh, den_sh,
         sgl0, sgl1, sgr0, sgr1, se0, se1, sss, ssc, srs0, srs1, sds) = rest
        xlv = [xlv0, xlv1]
        xrv = [xrv0, xrv1]
        ev = [ev0, ev1]
        rowv = xrv
        sgl = [sgl0, sgl1]
        sgr = [sgr0, sgr1]
        se = [se0, se1]
        srs = [srs0, srs1]

        c = lax.axis_index("c")
        s = lax.axis_index("s")
        wid = s * NC + c
        base0 = wid * EPW

        pltpu.sync_copy(att_hbm, attv)
        pltpu.sync_copy(src2_hbm.at[wid], srcv2)
        pltpu.sync_copy(dst2_hbm.at[wid], dstv2)

        ones16 = jnp.full((16,), 1.0, f32)

        def ofill(b, carry):
            onesv[pl.ds(16 * b, 16)] = ones16
            return carry

        lax.fori_loop(0, NG, ofill, 0)

        @pl.when(s == 0)
        def _():
            pltpu.sync_copy(zn_hbm, ssum_sh)
            pltpu.sync_copy(znh_hbm, acc_sh)
            pltpu.sync_copy(zn_hbm, den_sh)
            pltpu.sync_copy(zn_hbm, cnt_sh)

        plsc.subcore_barrier()

        attvec = attv[...]
        attks = [attvec[k] for k in range(H)]
        iota16 = lax.iota(jnp.int32, 16)
        kvecs = [jnp.full((16,), k, jnp.int32) for k in range(H)]

        def issue_g1(j, slot):
            pltpu.async_copy(xl_hbm.at[srcv2.at[j]], xlv[slot], sgl[slot])
            pltpu.async_copy(xr_hbm.at[dstv2.at[j]], xrv[slot], sgr[slot])
            pltpu.async_copy(el_hbm.at[pl.ds(base0 + j * C, C)],
                             ev[slot], se[slot])

        def process1(j, slot):
            pltpu.make_async_copy(xl_hbm.at[srcv2.at[j]], xlv[slot], sgl[slot]).wait()
            pltpu.make_async_copy(xr_hbm.at[dstv2.at[j]], xrv[slot], sgr[slot]).wait()
            pltpu.make_async_copy(el_hbm.at[pl.ds(base0 + j * C, C)],
                                  ev[slot], se[slot]).wait()
            def g1(b, carry):
                ivec = iota16 + b * 16
                acc = jnp.zeros((16,), f32)
                for k in range(H):
                    z = (plsc.load_gather(xlv[slot], [ivec, kvecs[k]])
                         + plsc.load_gather(xrv[slot], [ivec, kvecs[k]])
                         + plsc.load_gather(ev[slot], [ivec, kvecs[k]]))
                    m = jnp.maximum(z, 0.2 * z)
                    acc = acc + m * attks[k]
                avb[pl.ds(j * C + b * 16, 16)] = acc
                return carry

            lax.fori_loop(0, NG, g1, 0)
            pltpu.async_copy(avb.at[pl.ds(j * C, C)], ssum_sh.at[dstv2.at[j]],
                             sss, add=True)
            pltpu.async_copy(onesv, cnt_sh.at[dstv2.at[j]], ssc, add=True)

        issue_g1(0, 0)

        def pair1(t, carry):
            a = 2 * t
            issue_g1(a + 1, 1)
            process1(a, 0)
            issue_g1(a + 2, 0)
            process1(a + 1, 1)
            return carry

        lax.fori_loop(0, NCH // 2, pair1, 0)
        process1(NCH - 1, 0)

        def drain1(i, carry):
            pltpu.make_async_copy(avb.at[pl.ds(0, C)], ssum_sh.at[dstv2.at[0]],
                                  sss).wait()
            pltpu.make_async_copy(onesv, cnt_sh.at[dstv2.at[0]], ssc).wait()
            return carry

        lax.fori_loop(0, NCH, drain1, 0)

        plsc.subcore_barrier()

        pltpu.sync_copy(ssum_sh, t0v)
        pltpu.sync_copy(cnt_sh, c0v)

        def sbody(i, carry):
            sl = pl.ds(i * 16, 16)
            shift_v[sl] = t0v[sl] / jnp.maximum(c0v[sl], 1.0)
            return carry

        lax.fori_loop(0, N // 16, sbody, 0)

        @pl.when((s == 0) & (c == 0))
        def _():
            pltpu.sync_copy(ssum_sh, ssum0_hbm)
            pltpu.sync_copy(cnt_sh, cnt0_hbm)

        @pl.when((s == 0) & (c == 1))
        def _():
            pltpu.sync_copy(ssum_sh, ssum1_hbm)
            pltpu.sync_copy(cnt_sh, cnt1_hbm)

        def issue_g2(j, slot):
            pltpu.async_copy(xl_hbm.at[srcv2.at[j]], xlv[slot], sgl[slot])

        zero16 = jnp.zeros((16,), f32)
        for slot in range(2):
            def zfill(i, carry, _slot=slot):
                rowv[_slot][i, :] = zero16
                return carry

            lax.fori_loop(0, C, zfill, 0)
            pltpu.async_copy(rowv[slot], acc_sh.at[dstv2.at[0]], srs[slot],
                             add=True)

        issue_g2(0, 0)

        def process2(j, slot):
            pltpu.make_async_copy(xl_hbm.at[srcv2.at[j]], xlv[slot], sgl[slot]).wait()
            pltpu.make_async_copy(rowv[slot], acc_sh.at[dstv2.at[0]],
                                  srs[slot]).wait()
            def g2(b, carry):
                sl = pl.ds(j * C + b * 16, 16)
                dvec = dstv2[j, pl.ds(b * 16, 16)]
                svec = plsc.load_gather(shift_v, [dvec])
                exvec = jnp.exp(avb[sl] - svec)
                exb[sl] = exvec
                for t in range(16):
                    i = b * 16 + t
                    rowv[slot][i, :] = xlv[slot][i, :] * exvec[t]
                return carry

            lax.fori_loop(0, NG, g2, 0)
            pltpu.async_copy(exb.at[pl.ds(j * C, C)], den_sh.at[dstv2.at[j]],
                             sds, add=True)
            pltpu.async_copy(rowv[slot], acc_sh.at[dstv2.at[j]], srs[slot],
                             add=True)

        def pair2(t, carry):
            a = 2 * t
            issue_g2(a + 1, 1)
            process2(a, 0)
            issue_g2(a + 2, 0)
            process2(a + 1, 1)
            return carry

        lax.fori_loop(0, NCH // 2, pair2, 0)
        process2(NCH - 1, 0)

        def drain2(i, carry):
            pltpu.make_async_copy(exb.at[pl.ds(0, C)], den_sh.at[dstv2.at[0]],
                                  sds).wait()
            return carry

        lax.fori_loop(0, NCH, drain2, 0)
        pltpu.make_async_copy(rowv[0], acc_sh.at[dstv2.at[0]], srs[0]).wait()
        pltpu.make_async_copy(rowv[1], acc_sh.at[dstv2.at[0]], srs[1]).wait()

        plsc.subcore_barrier()

        @pl.when((s == 0) & (c == 0))
        def _():
            pltpu.sync_copy(acc_sh, acc0_hbm)
            pltpu.sync_copy(den_sh, den0_hbm)

        @pl.when((s == 0) & (c == 1))
        def _():
            pltpu.sync_copy(acc_sh, acc1_hbm)
            pltpu.sync_copy(den_sh, den1_hbm)

    out_type = [jax.ShapeDtypeStruct((N,), f32),
                jax.ShapeDtypeStruct((N,), f32),
                jax.ShapeDtypeStruct((N,), f32),
                jax.ShapeDtypeStruct((N,), f32),
                jax.ShapeDtypeStruct((N, H), f32),
                jax.ShapeDtypeStruct((N, H), f32),
                jax.ShapeDtypeStruct((N,), f32),
                jax.ShapeDtypeStruct((N,), f32)]
    scratch = [
        pltpu.VMEM((NCH, C), jnp.int32),
        pltpu.VMEM((NCH, C), jnp.int32),
        pltpu.VMEM((C, H), f32),
        pltpu.VMEM((C, H), f32),
        pltpu.VMEM((C, H), f32),
        pltpu.VMEM((C, H), f32),
        pltpu.VMEM((C, H), f32),
        pltpu.VMEM((C, H), f32),
        pltpu.VMEM((EPW,), f32),
        pltpu.VMEM((EPW,), f32),
        pltpu.VMEM((N,), f32),
        pltpu.VMEM((N,), f32),
        pltpu.VMEM((N,), f32),
        pltpu.VMEM((C,), f32),
        pltpu.VMEM((16,), f32),
        pltpu.VMEM_SHARED((N,), f32),
        pltpu.VMEM_SHARED((N,), f32),
        pltpu.VMEM_SHARED((N, H), f32),
        pltpu.VMEM_SHARED((N,), f32),
    ]
    scratch += [pltpu.SemaphoreType.DMA] * 11

    return pl.kernel(body, out_type=out_type, mesh=_mesh,
                     scratch_types=scratch, compiler_params=_sc_params)


_layer_sc = _make_layer_sc()


def kernel(x, edge_index, edge_attr, params, Wlin, blin):
    src = edge_index[0].astype(jnp.int32)
    dst = edge_index[1].astype(jnp.int32)
    src2 = src.reshape(NW, NCH, C)
    dst2 = dst.reshape(NW, NCH, C)

    we3 = jnp.stack([p[4] for p in params])
    els = _edge_emb(edge_attr, we3)

    zn = jnp.zeros((N,), f32)
    znh = jnp.zeros((N, H), f32)

    wl, bl, wr, br, _, att, bias = params[0]
    xl, xr = _proj(x, wl, bl.reshape(1, H), wr, br.reshape(1, H))

    out = None
    for l in range(3):
        s0, s1, c0, c1, a0, a1, d0, d1 = _layer_sc(
            src2, dst2, xl, xr, els[l], att, zn, znh)
        col = lambda v: v.reshape(N, 1)
        args = (a0, a1, col(d0), col(d1), col(s0), col(s1), col(c0), col(c1),
                bias.reshape(1, H))
        if l < 2:
            nwl, nbl, nwr, nbr, _, natt, nbias = params[l + 1]
            xl, xr = _combine_proj(*args, nwl, nbl.reshape(1, H),
                                   nwr, nbr.reshape(1, H))
            att = natt
            bias = nbias
        else:
            out = _final(*args, Wlin, blin.reshape(1, 1))
    return out


# speedup vs baseline: 1.3721x; 1.0613x over previous
import jax
import jax.numpy as jnp
from jax import lax
from jax.experimental import pallas as pl
from jax.experimental.pallas import tpu as pltpu
from jax.experimental.pallas import tpu_sc as plsc

N = 10000
E = 320000
D = 128
H = 16
DE = 16

NC = 2
NS = 16
NW = NC * NS
EPW = E // NW
C = 80
NCH = EPW // C
NG = C // 16

_mesh = plsc.VectorSubcoreMesh(
    core_axis_name="c", subcore_axis_name="s", num_cores=NC, num_subcores=NS)

f32 = jnp.float32


def _edge_emb_body(ea_ref, we_ref, o0_ref, o1_ref, o2_ref):
    ea = ea_ref[...]
    o0_ref[...] = jnp.dot(ea, we_ref[0], preferred_element_type=f32)
    o1_ref[...] = jnp.dot(ea, we_ref[1], preferred_element_type=f32)
    o2_ref[...] = jnp.dot(ea, we_ref[2], preferred_element_type=f32)


def _edge_emb(edge_attr, we3):
    EB = 4000
    return pl.pallas_call(
        _edge_emb_body,
        grid=(E // EB,),
        in_specs=[
            pl.BlockSpec((EB, DE), lambda i: (i, 0)),
            pl.BlockSpec((3, DE, H), lambda i: (0, 0, 0)),
        ],
        out_specs=[pl.BlockSpec((EB, H), lambda i: (i, 0))] * 3,
        out_shape=[jax.ShapeDtypeStruct((E, H), f32)] * 3,
    )(edge_attr, we3)


def _proj_body(x_ref, wl_ref, bl_ref, wr_ref, br_ref, xl_ref, xr_ref):
    xv = x_ref[...]
    xl_ref[...] = jnp.dot(xv, wl_ref[...], preferred_element_type=f32) + bl_ref[...]
    xr_ref[...] = jnp.dot(xv, wr_ref[...], preferred_element_type=f32) + br_ref[...]


def _proj(x, wl, bl, wr, br):
    NB = 2000
    din = x.shape[1]
    return pl.pallas_call(
        _proj_body,
        grid=(N // NB,),
        in_specs=[
            pl.BlockSpec((NB, din), lambda i: (i, 0)),
            pl.BlockSpec((din, H), lambda i: (0, 0)),
            pl.BlockSpec((H,), lambda i: (0,)),
            pl.BlockSpec((din, H), lambda i: (0, 0)),
            pl.BlockSpec((H,), lambda i: (0,)),
        ],
        out_specs=[
            pl.BlockSpec((NB, H), lambda i: (i, 0)),
            pl.BlockSpec((NB, H), lambda i: (i, 0)),
        ],
        out_shape=[
            jax.ShapeDtypeStruct((N, H), f32),
            jax.ShapeDtypeStruct((N, H), f32),
        ],
    )(x, wl, bl, wr, br)


def _softmax_h(a0, a1, scl, bias):
    sh0 = scl[:, 0:1] / jnp.maximum(scl[:, 1:2], 1.0)
    sh1 = scl[:, 3:4] / jnp.maximum(scl[:, 4:5], 1.0)
    sm = jnp.maximum(sh0, sh1)
    w0 = jnp.exp(sh0 - sm)
    w1 = jnp.exp(sh1 - sm)
    den = scl[:, 2:3] * w0 + scl[:, 5:6] * w1 + 1e-16
    acc = a0 * w0 + a1 * w1
    return acc / den + bias


_node_specs = [
    pl.BlockSpec((2000, H), lambda i: (i, 0)),
    pl.BlockSpec((2000, H), lambda i: (i, 0)),
    pl.BlockSpec((2000, 6), lambda i: (i, 0)),
    pl.BlockSpec((H,), lambda i: (0,)),
]


def _combine_proj_body(a0_ref, a1_ref, scl_ref, bias_ref,
                       wl_ref, bl_ref, wr_ref, br_ref, xl_ref, xr_ref):
    h = _softmax_h(a0_ref[...], a1_ref[...], scl_ref[...], bias_ref[...])
    xl_ref[...] = jnp.dot(h, wl_ref[...], preferred_element_type=f32) + bl_ref[...]
    xr_ref[...] = jnp.dot(h, wr_ref[...], preferred_element_type=f32) + br_ref[...]


def _combine_proj(a0, a1, scl, bias, wl, bl, wr, br):
    return pl.pallas_call(
        _combine_proj_body,
        grid=(5,),
        in_specs=_node_specs + [
            pl.BlockSpec((H, H), lambda i: (0, 0)),
            pl.BlockSpec((H,), lambda i: (0,)),
            pl.BlockSpec((H, H), lambda i: (0, 0)),
            pl.BlockSpec((H,), lambda i: (0,)),
        ],
        out_specs=[
            pl.BlockSpec((2000, H), lambda i: (i, 0)),
            pl.BlockSpec((2000, H), lambda i: (i, 0)),
        ],
        out_shape=[
            jax.ShapeDtypeStruct((N, H), f32),
            jax.ShapeDtypeStruct((N, H), f32),
        ],
    )(a0, a1, scl, bias, wl, bl, wr, br)


def _final_body(a0_ref, a1_ref, scl_ref, bias_ref, wlin_ref, blin_ref, out_ref):
    i = pl.program_id(0)
    h = _softmax_h(a0_ref[...], a1_ref[...], scl_ref[...], bias_ref[...])
    part = jnp.sum(jnp.dot(h, wlin_ref[...], preferred_element_type=f32))

    @pl.when(i == 0)
    def _():
        out_ref[...] = jnp.zeros((1, 1), f32)

    out_ref[...] += jnp.reshape(part / N, (1, 1))

    @pl.when(i == pl.num_programs(0) - 1)
    def _():
        out_ref[...] += jnp.reshape(blin_ref[...], (1, 1))


def _final(a0, a1, scl, bias, wlin, blin):
    return pl.pallas_call(
        _final_body,
        grid=(5,),
        in_specs=_node_specs + [
            pl.BlockSpec((H, 1), lambda i: (0, 0)),
            pl.BlockSpec((1,), lambda i: (0,)),
        ],
        out_specs=pl.BlockSpec((1, 1), lambda i: (0, 0)),
        out_shape=jax.ShapeDtypeStruct((1, 1), f32),
    )(a0, a1, scl, bias, wlin, blin)


_sc_params = pltpu.CompilerParams(
    needs_layout_passes=False, use_tc_tiling_on_sc=False)


def _make_layer_sc():
    def body(src2_hbm, dst2_hbm, xl_hbm, xr_hbm, el_hbm, att_hbm,
             zn_hbm, znh_hbm, *rest):
        (ssum0_hbm, ssum1_hbm, cnt0_hbm, cnt1_hbm,
         acc0_hbm, acc1_hbm, den0_hbm, den1_hbm,
         srcv2, dstv2, xlv0, xlv1, xrv0, xrv1, ev0, ev1,
         avb, exb, shift_v, t0v, c0v, onesv, attv,
         ssum_sh, cnt_sh, acc_sh, den_sh,
         sgl0, sgl1, sgr0, sgr1, se0, se1, sss, ssc, srs0, srs1, sds) = rest
        xlv = [xlv0, xlv1]
        xrv = [xrv0, xrv1]
        ev = [ev0, ev1]
        rowv = xrv
        sgl = [sgl0, sgl1]
        sgr = [sgr0, sgr1]
        se = [se0, se1]
        srs = [srs0, srs1]

        c = lax.axis_index("c")
        s = lax.axis_index("s")
        wid = s * NC + c
        base0 = wid * EPW

        pltpu.sync_copy(att_hbm, attv)
        pltpu.sync_copy(src2_hbm.at[wid], srcv2)
        pltpu.sync_copy(dst2_hbm.at[wid], dstv2)

        ones16 = jnp.full((16,), 1.0, f32)

        def ofill(b, carry):
            onesv[pl.ds(16 * b, 16)] = ones16
            return carry

        lax.fori_loop(0, NG, ofill, 0)

        @pl.when(s == 0)
        def _():
            pltpu.sync_copy(zn_hbm, ssum_sh)
            pltpu.sync_copy(znh_hbm, acc_sh)
            pltpu.sync_copy(zn_hbm, den_sh)
            pltpu.sync_copy(zn_hbm, cnt_sh)

        plsc.subcore_barrier()

        attvec = attv[...]
        attks = [attvec[k] for k in range(H)]
        iota16 = lax.iota(jnp.int32, 16)
        kvecs = [jnp.full((16,), k, jnp.int32) for k in range(H)]

        def issue_g1(j, slot):
            pltpu.async_copy(xl_hbm.at[srcv2.at[j]], xlv[slot], sgl[slot])
            pltpu.async_copy(xr_hbm.at[dstv2.at[j]], xrv[slot], sgr[slot])
            pltpu.async_copy(el_hbm.at[pl.ds(base0 + j * C, C)],
                             ev[slot], se[slot])

        def process1(j, slot):
            pltpu.make_async_copy(xl_hbm.at[srcv2.at[j]], xlv[slot], sgl[slot]).wait()
            pltpu.make_async_copy(xr_hbm.at[dstv2.at[j]], xrv[slot], sgr[slot]).wait()
            pltpu.make_async_copy(el_hbm.at[pl.ds(base0 + j * C, C)],
                                  ev[slot], se[slot]).wait()
            def g1(b, carry):
                ivec = iota16 + b * 16
                acc = jnp.zeros((16,), f32)
                for k in range(H):
                    z = (plsc.load_gather(xlv[slot], [ivec, kvecs[k]])
                         + plsc.load_gather(xrv[slot], [ivec, kvecs[k]])
                         + plsc.load_gather(ev[slot], [ivec, kvecs[k]]))
                    m = jnp.maximum(z, 0.2 * z)
                    acc = acc + m * attks[k]
                avb[pl.ds(j * C + b * 16, 16)] = acc
                return carry

            lax.fori_loop(0, NG, g1, 0)
            pltpu.async_copy(avb.at[pl.ds(j * C, C)], ssum_sh.at[dstv2.at[j]],
                             sss, add=True)
            pltpu.async_copy(onesv, cnt_sh.at[dstv2.at[j]], ssc, add=True)

        issue_g1(0, 0)

        def pair1(t, carry):
            a = 2 * t
            issue_g1(a + 1, 1)
            process1(a, 0)
            issue_g1(a + 2, 0)
            process1(a + 1, 1)
            return carry

        lax.fori_loop(0, NCH // 2, pair1, 0)
        process1(NCH - 1, 0)

        def drain1(i, carry):
            pltpu.make_async_copy(avb.at[pl.ds(0, C)], ssum_sh.at[dstv2.at[0]],
                                  sss).wait()
            pltpu.make_async_copy(onesv, cnt_sh.at[dstv2.at[0]], ssc).wait()
            return carry

        lax.fori_loop(0, NCH, drain1, 0)

        plsc.subcore_barrier()

        pltpu.sync_copy(ssum_sh, t0v)
        pltpu.sync_copy(cnt_sh, c0v)

        def sbody(i, carry):
            sl = pl.ds(i * 16, 16)
            shift_v[sl] = t0v[sl] / jnp.maximum(c0v[sl], 1.0)
            return carry

        lax.fori_loop(0, N // 16, sbody, 0)

        @pl.when((s == 0) & (c == 0))
        def _():
            pltpu.sync_copy(ssum_sh, ssum0_hbm)
            pltpu.sync_copy(cnt_sh, cnt0_hbm)

        @pl.when((s == 0) & (c == 1))
        def _():
            pltpu.sync_copy(ssum_sh, ssum1_hbm)
            pltpu.sync_copy(cnt_sh, cnt1_hbm)

        def issue_g2(j, slot):
            pltpu.async_copy(xl_hbm.at[srcv2.at[j]], xlv[slot], sgl[slot])

        zero16 = jnp.zeros((16,), f32)
        for slot in range(2):
            def zfill(i, carry, _slot=slot):
                rowv[_slot][i, :] = zero16
                return carry

            lax.fori_loop(0, C, zfill, 0)
            pltpu.async_copy(rowv[slot], acc_sh.at[dstv2.at[0]], srs[slot],
                             add=True)

        issue_g2(0, 0)

        def process2(j, slot):
            pltpu.make_async_copy(xl_hbm.at[srcv2.at[j]], xlv[slot], sgl[slot]).wait()
            pltpu.make_async_copy(rowv[slot], acc_sh.at[dstv2.at[0]],
                                  srs[slot]).wait()
            def g2(b, carry):
                sl = pl.ds(j * C + b * 16, 16)
                dvec = dstv2[j, pl.ds(b * 16, 16)]
                svec = plsc.load_gather(shift_v, [dvec])
                exvec = jnp.exp(avb[sl] - svec)
                exb[sl] = exvec
                for t in range(16):
                    i = b * 16 + t
                    rowv[slot][i, :] = xlv[slot][i, :] * exvec[t]
                return carry

            lax.fori_loop(0, NG, g2, 0)
            pltpu.async_copy(exb.at[pl.ds(j * C, C)], den_sh.at[dstv2.at[j]],
                             sds, add=True)
            pltpu.async_copy(rowv[slot], acc_sh.at[dstv2.at[j]], srs[slot],
                             add=True)

        def pair2(t, carry):
            a = 2 * t
            issue_g2(a + 1, 1)
            process2(a, 0)
            issue_g2(a + 2, 0)
            process2(a + 1, 1)
            return carry

        lax.fori_loop(0, NCH // 2, pair2, 0)
        process2(NCH - 1, 0)

        def drain2(i, carry):
            pltpu.make_async_copy(exb.at[pl.ds(0, C)], den_sh.at[dstv2.at[0]],
                                  sds).wait()
            return carry

        lax.fori_loop(0, NCH, drain2, 0)
        pltpu.make_async_copy(rowv[0], acc_sh.at[dstv2.at[0]], srs[0]).wait()
        pltpu.make_async_copy(rowv[1], acc_sh.at[dstv2.at[0]], srs[1]).wait()

        plsc.subcore_barrier()

        @pl.when((s == 0) & (c == 0))
        def _():
            pltpu.sync_copy(acc_sh, acc0_hbm)
            pltpu.sync_copy(den_sh, den0_hbm)

        @pl.when((s == 0) & (c == 1))
        def _():
            pltpu.sync_copy(acc_sh, acc1_hbm)
            pltpu.sync_copy(den_sh, den1_hbm)

    out_type = [jax.ShapeDtypeStruct((N,), f32),
                jax.ShapeDtypeStruct((N,), f32),
                jax.ShapeDtypeStruct((N,), f32),
                jax.ShapeDtypeStruct((N,), f32),
                jax.ShapeDtypeStruct((N, H), f32),
                jax.ShapeDtypeStruct((N, H), f32),
                jax.ShapeDtypeStruct((N,), f32),
                jax.ShapeDtypeStruct((N,), f32)]
    scratch = [
        pltpu.VMEM((NCH, C), jnp.int32),
        pltpu.VMEM((NCH, C), jnp.int32),
        pltpu.VMEM((C, H), f32),
        pltpu.VMEM((C, H), f32),
        pltpu.VMEM((C, H), f32),
        pltpu.VMEM((C, H), f32),
        pltpu.VMEM((C, H), f32),
        pltpu.VMEM((C, H), f32),
        pltpu.VMEM((EPW,), f32),
        pltpu.VMEM((EPW,), f32),
        pltpu.VMEM((N,), f32),
        pltpu.VMEM((N,), f32),
        pltpu.VMEM((N,), f32),
        pltpu.VMEM((C,), f32),
        pltpu.VMEM((16,), f32),
        pltpu.VMEM_SHARED((N,), f32),
        pltpu.VMEM_SHARED((N,), f32),
        pltpu.VMEM_SHARED((N, H), f32),
        pltpu.VMEM_SHARED((N,), f32),
    ]
    scratch += [pltpu.SemaphoreType.DMA] * 11

    return pl.kernel(body, out_type=out_type, mesh=_mesh,
                     scratch_types=scratch, compiler_params=_sc_params)


_layer_sc = _make_layer_sc()


def kernel(x, edge_index, edge_attr, params, Wlin, blin):
    src = edge_index[0].astype(jnp.int32)
    dst = edge_index[1].astype(jnp.int32)
    src2 = src.reshape(NW, NCH, C)
    dst2 = dst.reshape(NW, NCH, C)

    we3 = jnp.stack([p[4] for p in params])
    els = _edge_emb(edge_attr, we3)

    zn = jnp.zeros((N,), f32)
    znh = jnp.zeros((N, H), f32)

    wl, bl, wr, br, _, att, bias = params[0]
    xl, xr = _proj(x, wl, bl, wr, br)

    out = None
    for l in range(3):
        s0, s1, c0, c1, a0, a1, d0, d1 = _layer_sc(
            src2, dst2, xl, xr, els[l], att, zn, znh)
        scl = jnp.stack([s0, c0, d0, s1, c1, d1], axis=1)
        args = (a0, a1, scl, bias)
        if l < 2:
            nwl, nbl, nwr, nbr, _, natt, nbias = params[l + 1]
            xl, xr = _combine_proj(*args, nwl, nbl, nwr, nbr)
            att = natt
            bias = nbias
        else:
            out = _final(*args, Wlin, blin)
    return out


# speedup vs baseline: 1.4954x; 1.0899x over previous
import jax
import jax.numpy as jnp
from jax import lax
from jax.experimental import pallas as pl
from jax.experimental.pallas import tpu as pltpu
from jax.experimental.pallas import tpu_sc as plsc

N = 10000
E = 320000
D = 128
H = 16
DE = 16

NC = 2
NS = 16
NW = NC * NS
EPW = E // NW
C = 80
NCH = EPW // C
NG = C // 16

_mesh = plsc.VectorSubcoreMesh(
    core_axis_name="c", subcore_axis_name="s", num_cores=NC, num_subcores=NS)

f32 = jnp.float32


def _edge_emb_body(ea_ref, we_ref, o0_ref, o1_ref, o2_ref):
    ea = ea_ref[...]
    o0_ref[...] = jnp.dot(ea, we_ref[0], preferred_element_type=f32)
    o1_ref[...] = jnp.dot(ea, we_ref[1], preferred_element_type=f32)
    o2_ref[...] = jnp.dot(ea, we_ref[2], preferred_element_type=f32)


def _edge_emb(edge_attr, we3):
    EB = 4000
    return pl.pallas_call(
        _edge_emb_body,
        grid=(E // EB,),
        in_specs=[
            pl.BlockSpec((EB, DE), lambda i: (i, 0)),
            pl.BlockSpec((3, DE, H), lambda i: (0, 0, 0)),
        ],
        out_specs=[pl.BlockSpec((EB, H), lambda i: (i, 0))] * 3,
        out_shape=[jax.ShapeDtypeStruct((E, H), f32)] * 3,
    )(edge_attr, we3)


def _proj_body(x_ref, wl_ref, bl_ref, wr_ref, br_ref, xl_ref, xr_ref):
    xv = x_ref[...]
    xl_ref[...] = jnp.dot(xv, wl_ref[...], preferred_element_type=f32) + bl_ref[...]
    xr_ref[...] = jnp.dot(xv, wr_ref[...], preferred_element_type=f32) + br_ref[...]


def _proj(x, wl, bl, wr, br):
    NB = 2000
    din = x.shape[1]
    return pl.pallas_call(
        _proj_body,
        grid=(N // NB,),
        in_specs=[
            pl.BlockSpec((NB, din), lambda i: (i, 0)),
            pl.BlockSpec((din, H), lambda i: (0, 0)),
            pl.BlockSpec((H,), lambda i: (0,)),
            pl.BlockSpec((din, H), lambda i: (0, 0)),
            pl.BlockSpec((H,), lambda i: (0,)),
        ],
        out_specs=[
            pl.BlockSpec((NB, H), lambda i: (i, 0)),
            pl.BlockSpec((NB, H), lambda i: (i, 0)),
        ],
        out_shape=[
            jax.ShapeDtypeStruct((N, H), f32),
            jax.ShapeDtypeStruct((N, H), f32),
        ],
    )(x, wl, bl, wr, br)


def _softmax_h(a0, a1, scl, bias):
    sh0 = scl[:, 0:1] / jnp.maximum(scl[:, 1:2], 1.0)
    sh1 = scl[:, 3:4] / jnp.maximum(scl[:, 4:5], 1.0)
    sm = jnp.maximum(sh0, sh1)
    w0 = jnp.exp(sh0 - sm)
    w1 = jnp.exp(sh1 - sm)
    den = scl[:, 2:3] * w0 + scl[:, 5:6] * w1 + 1e-16
    acc = a0 * w0 + a1 * w1
    return acc / den + bias


_node_specs = [
    pl.BlockSpec((2000, H), lambda i: (i, 0)),
    pl.BlockSpec((2000, H), lambda i: (i, 0)),
    pl.BlockSpec((2000, 6), lambda i: (i, 0)),
    pl.BlockSpec((H,), lambda i: (0,)),
]


def _combine_proj_body(a0_ref, a1_ref, scl_ref, bias_ref,
                       wl_ref, bl_ref, wr_ref, br_ref, xl_ref, xr_ref):
    h = _softmax_h(a0_ref[...], a1_ref[...], scl_ref[...], bias_ref[...])
    xl_ref[...] = jnp.dot(h, wl_ref[...], preferred_element_type=f32) + bl_ref[...]
    xr_ref[...] = jnp.dot(h, wr_ref[...], preferred_element_type=f32) + br_ref[...]


def _combine_proj(a0, a1, scl, bias, wl, bl, wr, br):
    return pl.pallas_call(
        _combine_proj_body,
        grid=(5,),
        in_specs=_node_specs + [
            pl.BlockSpec((H, H), lambda i: (0, 0)),
            pl.BlockSpec((H,), lambda i: (0,)),
            pl.BlockSpec((H, H), lambda i: (0, 0)),
            pl.BlockSpec((H,), lambda i: (0,)),
        ],
        out_specs=[
            pl.BlockSpec((2000, H), lambda i: (i, 0)),
            pl.BlockSpec((2000, H), lambda i: (i, 0)),
        ],
        out_shape=[
            jax.ShapeDtypeStruct((N, H), f32),
            jax.ShapeDtypeStruct((N, H), f32),
        ],
    )(a0, a1, scl, bias, wl, bl, wr, br)


def _final_body(a0_ref, a1_ref, scl_ref, bias_ref, wlin_ref, blin_ref, out_ref):
    i = pl.program_id(0)
    h = _softmax_h(a0_ref[...], a1_ref[...], scl_ref[...], bias_ref[...])
    part = jnp.sum(jnp.dot(h, wlin_ref[...], preferred_element_type=f32))

    @pl.when(i == 0)
    def _():
        out_ref[...] = jnp.zeros((1, 1), f32)

    out_ref[...] += jnp.reshape(part / N, (1, 1))

    @pl.when(i == pl.num_programs(0) - 1)
    def _():
        out_ref[...] += jnp.reshape(blin_ref[...], (1, 1))


def _final(a0, a1, scl, bias, wlin, blin):
    return pl.pallas_call(
        _final_body,
        grid=(5,),
        in_specs=_node_specs + [
            pl.BlockSpec((H, 1), lambda i: (0, 0)),
            pl.BlockSpec((1,), lambda i: (0,)),
        ],
        out_specs=pl.BlockSpec((1, 1), lambda i: (0, 0)),
        out_shape=jax.ShapeDtypeStruct((1, 1), f32),
    )(a0, a1, scl, bias, wlin, blin)


_sc_params = pltpu.CompilerParams(
    needs_layout_passes=False, use_tc_tiling_on_sc=False)


def _make_layer_sc():
    def body(src2_hbm, dst2_hbm, xl_hbm, xr_hbm, el_hbm, att_hbm,
             zn_hbm, znh_hbm, *rest):
        (ssum0_hbm, ssum1_hbm, cnt0_hbm, cnt1_hbm,
         acc0_hbm, acc1_hbm, den0_hbm, den1_hbm,
         srcv2, dstv2,
         xlv0, xlv1, xlv2, xlv3, xrv0, xrv1, xrv2, xrv3,
         ev0, ev1, ev2, ev3,
         avb, exb, shift_v, t0v, c0v, onesv, attv,
         ssum_sh, cnt_sh, acc_sh, den_sh,
         sgl0, sgl1, sgl2, sgl3, sgr0, sgr1, sgr2, sgr3,
         se0, se1, se2, se3, sss, ssc,
         srs0, srs1, srs2, srs3, sds) = rest
        xlv = [xlv0, xlv1, xlv2, xlv3]
        xrv = [xrv0, xrv1, xrv2, xrv3]
        ev = [ev0, ev1, ev2, ev3]
        rowv = xrv
        sgl = [sgl0, sgl1, sgl2, sgl3]
        sgr = [sgr0, sgr1, sgr2, sgr3]
        se = [se0, se1, se2, se3]
        srs = [srs0, srs1, srs2, srs3]

        c = lax.axis_index("c")
        s = lax.axis_index("s")
        wid = s * NC + c
        base0 = wid * EPW

        pltpu.sync_copy(att_hbm, attv)
        pltpu.sync_copy(src2_hbm.at[wid], srcv2)
        pltpu.sync_copy(dst2_hbm.at[wid], dstv2)

        ones16 = jnp.full((16,), 1.0, f32)

        def ofill(b, carry):
            onesv[pl.ds(16 * b, 16)] = ones16
            return carry

        lax.fori_loop(0, NG, ofill, 0)

        @pl.when(s == 0)
        def _():
            pltpu.sync_copy(zn_hbm, ssum_sh)
            pltpu.sync_copy(znh_hbm, acc_sh)
            pltpu.sync_copy(zn_hbm, den_sh)
            pltpu.sync_copy(zn_hbm, cnt_sh)

        plsc.subcore_barrier()

        attvec = attv[...]
        attks = [attvec[k] for k in range(H)]
        iota16 = lax.iota(jnp.int32, 16)
        kvecs = [jnp.full((16,), k, jnp.int32) for k in range(H)]

        def issue_g1(j, slot):
            pltpu.async_copy(xl_hbm.at[srcv2.at[j]], xlv[slot], sgl[slot])
            pltpu.async_copy(xr_hbm.at[dstv2.at[j]], xrv[slot], sgr[slot])
            pltpu.async_copy(el_hbm.at[pl.ds(base0 + j * C, C)],
                             ev[slot], se[slot])

        def process1(j, slot):
            pltpu.make_async_copy(xl_hbm.at[srcv2.at[j]], xlv[slot], sgl[slot]).wait()
            pltpu.make_async_copy(xr_hbm.at[dstv2.at[j]], xrv[slot], sgr[slot]).wait()
            pltpu.make_async_copy(el_hbm.at[pl.ds(base0 + j * C, C)],
                                  ev[slot], se[slot]).wait()
            def g1(b, carry):
                ivec = iota16 + b * 16
                acc = jnp.zeros((16,), f32)
                for k in range(H):
                    z = (plsc.load_gather(xlv[slot], [ivec, kvecs[k]])
                         + plsc.load_gather(xrv[slot], [ivec, kvecs[k]])
                         + plsc.load_gather(ev[slot], [ivec, kvecs[k]]))
                    m = jnp.maximum(z, 0.2 * z)
                    acc = acc + m * attks[k]
                avb[pl.ds(j * C + b * 16, 16)] = acc
                return carry

            lax.fori_loop(0, NG, g1, 0)
            pltpu.async_copy(avb.at[pl.ds(j * C, C)], ssum_sh.at[dstv2.at[j]],
                             sss, add=True)
            pltpu.async_copy(onesv, cnt_sh.at[dstv2.at[j]], ssc, add=True)

        issue_g1(0, 0)
        issue_g1(1, 1)
        issue_g1(2, 2)

        def quad1(t, carry):
            for q in range(4):
                j = 4 * t + q
                process1(j, q)
                jn = j + 3

                @pl.when(jn < NCH)
                def _(jn=jn, q=q):
                    issue_g1(jn, (q + 3) % 4)
            return carry

        lax.fori_loop(0, NCH // 4, quad1, 0)
        process1(NCH - 1, 0)

        def drain1(i, carry):
            pltpu.make_async_copy(avb.at[pl.ds(0, C)], ssum_sh.at[dstv2.at[0]],
                                  sss).wait()
            pltpu.make_async_copy(onesv, cnt_sh.at[dstv2.at[0]], ssc).wait()
            return carry

        lax.fori_loop(0, NCH, drain1, 0)

        plsc.subcore_barrier()

        pltpu.sync_copy(ssum_sh, t0v)
        pltpu.sync_copy(cnt_sh, c0v)

        def sbody(i, carry):
            sl = pl.ds(i * 16, 16)
            shift_v[sl] = t0v[sl] / jnp.maximum(c0v[sl], 1.0)
            return carry

        lax.fori_loop(0, N // 16, sbody, 0)

        @pl.when((s == 0) & (c == 0))
        def _():
            pltpu.sync_copy(ssum_sh, ssum0_hbm)
            pltpu.sync_copy(cnt_sh, cnt0_hbm)

        @pl.when((s == 0) & (c == 1))
        def _():
            pltpu.sync_copy(ssum_sh, ssum1_hbm)
            pltpu.sync_copy(cnt_sh, cnt1_hbm)

        def issue_g2(j, slot):
            pltpu.async_copy(xl_hbm.at[srcv2.at[j]], xlv[slot], sgl[slot])

        zero16 = jnp.zeros((16,), f32)
        for slot in range(4):
            def zfill(i, carry, _slot=slot):
                rowv[_slot][i, :] = zero16
                return carry

            lax.fori_loop(0, C, zfill, 0)
            pltpu.async_copy(rowv[slot], acc_sh.at[dstv2.at[0]], srs[slot],
                             add=True)

        issue_g2(0, 0)
        issue_g2(1, 1)
        issue_g2(2, 2)

        def process2(j, slot):
            pltpu.make_async_copy(xl_hbm.at[srcv2.at[j]], xlv[slot], sgl[slot]).wait()
            pltpu.make_async_copy(rowv[slot], acc_sh.at[dstv2.at[0]],
                                  srs[slot]).wait()
            def g2(b, carry):
                sl = pl.ds(j * C + b * 16, 16)
                dvec = dstv2[j, pl.ds(b * 16, 16)]
                svec = plsc.load_gather(shift_v, [dvec])
                exvec = jnp.exp(avb[sl] - svec)
                exb[sl] = exvec
                for t in range(16):
                    i = b * 16 + t
                    rowv[slot][i, :] = xlv[slot][i, :] * exvec[t]
                return carry

            lax.fori_loop(0, NG, g2, 0)
            pltpu.async_copy(exb.at[pl.ds(j * C, C)], den_sh.at[dstv2.at[j]],
                             sds, add=True)
            pltpu.async_copy(rowv[slot], acc_sh.at[dstv2.at[j]], srs[slot],
                             add=True)

        def quad2(t, carry):
            for q in range(4):
                j = 4 * t + q
                process2(j, q)
                jn = j + 3

                @pl.when(jn < NCH)
                def _(jn=jn, q=q):
                    issue_g2(jn, (q + 3) % 4)
            return carry

        lax.fori_loop(0, NCH // 4, quad2, 0)
        process2(NCH - 1, 0)

        def drain2(i, carry):
            pltpu.make_async_copy(exb.at[pl.ds(0, C)], den_sh.at[dstv2.at[0]],
                                  sds).wait()
            return carry

        lax.fori_loop(0, NCH, drain2, 0)
        for slot in range(4):
            pltpu.make_async_copy(rowv[slot], acc_sh.at[dstv2.at[0]],
                                  srs[slot]).wait()

        plsc.subcore_barrier()

        @pl.when((s == 0) & (c == 0))
        def _():
            pltpu.sync_copy(acc_sh, acc0_hbm)
            pltpu.sync_copy(den_sh, den0_hbm)

        @pl.when((s == 0) & (c == 1))
        def _():
            pltpu.sync_copy(acc_sh, acc1_hbm)
            pltpu.sync_copy(den_sh, den1_hbm)

    out_type = [jax.ShapeDtypeStruct((N,), f32),
                jax.ShapeDtypeStruct((N,), f32),
                jax.ShapeDtypeStruct((N,), f32),
                jax.ShapeDtypeStruct((N,), f32),
                jax.ShapeDtypeStruct((N, H), f32),
                jax.ShapeDtypeStruct((N, H), f32),
                jax.ShapeDtypeStruct((N,), f32),
                jax.ShapeDtypeStruct((N,), f32)]
    scratch = [
        pltpu.VMEM((NCH, C), jnp.int32),
        pltpu.VMEM((NCH, C), jnp.int32),
        pltpu.VMEM((C, H), f32),
        pltpu.VMEM((C, H), f32),
        pltpu.VMEM((C, H), f32),
        pltpu.VMEM((C, H), f32),
        pltpu.VMEM((C, H), f32),
        pltpu.VMEM((C, H), f32),
        pltpu.VMEM((C, H), f32),
        pltpu.VMEM((C, H), f32),
        pltpu.VMEM((C, H), f32),
        pltpu.VMEM((C, H), f32),
        pltpu.VMEM((C, H), f32),
        pltpu.VMEM((C, H), f32),
        pltpu.VMEM((EPW,), f32),
        pltpu.VMEM((EPW,), f32),
        pltpu.VMEM((N,), f32),
        pltpu.VMEM((N,), f32),
        pltpu.VMEM((N,), f32),
        pltpu.VMEM((C,), f32),
        pltpu.VMEM((16,), f32),
        pltpu.VMEM_SHARED((N,), f32),
        pltpu.VMEM_SHARED((N,), f32),
        pltpu.VMEM_SHARED((N, H), f32),
        pltpu.VMEM_SHARED((N,), f32),
    ]
    scratch += [pltpu.SemaphoreType.DMA] * 19

    return pl.kernel(body, out_type=out_type, mesh=_mesh,
                     scratch_types=scratch, compiler_params=_sc_params)


_layer_sc = _make_layer_sc()


def kernel(x, edge_index, edge_attr, params, Wlin, blin):
    src = edge_index[0].astype(jnp.int32)
    dst = edge_index[1].astype(jnp.int32)
    src2 = src.reshape(NW, NCH, C)
    dst2 = dst.reshape(NW, NCH, C)

    we3 = jnp.stack([p[4] for p in params])
    els = _edge_emb(edge_attr, we3)

    zn = jnp.zeros((N,), f32)
    znh = jnp.zeros((N, H), f32)

    wl, bl, wr, br, _, att, bias = params[0]
    xl, xr = _proj(x, wl, bl, wr, br)

    out = None
    for l in range(3):
        s0, s1, c0, c1, a0, a1, d0, d1 = _layer_sc(
            src2, dst2, xl, xr, els[l], att, zn, znh)
        scl = jnp.stack([s0, c0, d0, s1, c1, d1], axis=1)
        args = (a0, a1, scl, bias)
        if l < 2:
            nwl, nbl, nwr, nbr, _, natt, nbias = params[l + 1]
            xl, xr = _combine_proj(*args, nwl, nbl, nwr, nbr)
            att = natt
            bias = nbias
        else:
            out = _final(*args, Wlin, blin)
    return out


# speedup vs baseline: 1.5138x; 1.0123x over previous
import jax
import jax.numpy as jnp
from jax import lax
from jax.experimental import pallas as pl
from jax.experimental.pallas import tpu as pltpu
from jax.experimental.pallas import tpu_sc as plsc

N = 10000
E = 320000
D = 128
H = 16
DE = 16

NC = 2
NS = 16
NW = NC * NS
EPW = E // NW
C = 80
NCH = EPW // C
NG = C // 16

_mesh = plsc.VectorSubcoreMesh(
    core_axis_name="c", subcore_axis_name="s", num_cores=NC, num_subcores=NS)

f32 = jnp.float32


def _edge_emb_body(ea_ref, we_ref, o0_ref, o1_ref, o2_ref):
    ea = ea_ref[...]
    o0_ref[...] = jnp.dot(ea, we_ref[0], preferred_element_type=f32)
    o1_ref[...] = jnp.dot(ea, we_ref[1], preferred_element_type=f32)
    o2_ref[...] = jnp.dot(ea, we_ref[2], preferred_element_type=f32)


def _edge_emb(edge_attr, we3):
    EB = 4000
    return pl.pallas_call(
        _edge_emb_body,
        grid=(E // EB,),
        in_specs=[
            pl.BlockSpec((EB, DE), lambda i: (i, 0)),
            pl.BlockSpec((3, DE, H), lambda i: (0, 0, 0)),
        ],
        out_specs=[pl.BlockSpec((EB, H), lambda i: (i, 0))] * 3,
        out_shape=[jax.ShapeDtypeStruct((E, H), f32)] * 3,
    )(edge_attr, we3)


def _proj_body(x_ref, wl_ref, bl_ref, wr_ref, br_ref, xl_ref, xr_ref):
    xv = x_ref[...]
    xl_ref[...] = jnp.dot(xv, wl_ref[...], preferred_element_type=f32) + bl_ref[...]
    xr_ref[...] = jnp.dot(xv, wr_ref[...], preferred_element_type=f32) + br_ref[...]


def _proj(x, wl, bl, wr, br):
    NB = 2000
    din = x.shape[1]
    return pl.pallas_call(
        _proj_body,
        grid=(N // NB,),
        in_specs=[
            pl.BlockSpec((NB, din), lambda i: (i, 0)),
            pl.BlockSpec((din, H), lambda i: (0, 0)),
            pl.BlockSpec((H,), lambda i: (0,)),
            pl.BlockSpec((din, H), lambda i: (0, 0)),
            pl.BlockSpec((H,), lambda i: (0,)),
        ],
        out_specs=[
            pl.BlockSpec((NB, H), lambda i: (i, 0)),
            pl.BlockSpec((NB, H), lambda i: (i, 0)),
        ],
        out_shape=[
            jax.ShapeDtypeStruct((N, H), f32),
            jax.ShapeDtypeStruct((N, H), f32),
        ],
    )(x, wl, bl, wr, br)


def _softmax_h(a0, a1, scl, bias):
    sh0 = scl[:, 0:1] / jnp.maximum(scl[:, 1:2], 1.0)
    sh1 = scl[:, 3:4] / jnp.maximum(scl[:, 4:5], 1.0)
    sm = jnp.maximum(sh0, sh1)
    w0 = jnp.exp(sh0 - sm)
    w1 = jnp.exp(sh1 - sm)
    den = scl[:, 2:3] * w0 + scl[:, 5:6] * w1 + 1e-16
    acc = a0 * w0 + a1 * w1
    return acc / den + bias


_node_specs = [
    pl.BlockSpec((2000, H), lambda i: (i, 0)),
    pl.BlockSpec((2000, H), lambda i: (i, 0)),
    pl.BlockSpec((2000, 6), lambda i: (i, 0)),
    pl.BlockSpec((H,), lambda i: (0,)),
]


def _combine_proj_body(a0_ref, a1_ref, scl_ref, bias_ref,
                       wl_ref, bl_ref, wr_ref, br_ref, xl_ref, xr_ref):
    h = _softmax_h(a0_ref[...], a1_ref[...], scl_ref[...], bias_ref[...])
    xl_ref[...] = jnp.dot(h, wl_ref[...], preferred_element_type=f32) + bl_ref[...]
    xr_ref[...] = jnp.dot(h, wr_ref[...], preferred_element_type=f32) + br_ref[...]


def _combine_proj(a0, a1, scl, bias, wl, bl, wr, br):
    return pl.pallas_call(
        _combine_proj_body,
        grid=(5,),
        in_specs=_node_specs + [
            pl.BlockSpec((H, H), lambda i: (0, 0)),
            pl.BlockSpec((H,), lambda i: (0,)),
            pl.BlockSpec((H, H), lambda i: (0, 0)),
            pl.BlockSpec((H,), lambda i: (0,)),
        ],
        out_specs=[
            pl.BlockSpec((2000, H), lambda i: (i, 0)),
            pl.BlockSpec((2000, H), lambda i: (i, 0)),
        ],
        out_shape=[
            jax.ShapeDtypeStruct((N, H), f32),
            jax.ShapeDtypeStruct((N, H), f32),
        ],
    )(a0, a1, scl, bias, wl, bl, wr, br)


def _final_body(a0_ref, a1_ref, scl_ref, bias_ref, wlin_ref, blin_ref, out_ref):
    i = pl.program_id(0)
    h = _softmax_h(a0_ref[...], a1_ref[...], scl_ref[...], bias_ref[...])
    part = jnp.sum(jnp.dot(h, wlin_ref[...], preferred_element_type=f32))

    @pl.when(i == 0)
    def _():
        out_ref[...] = jnp.zeros((1, 1), f32)

    out_ref[...] += jnp.reshape(part / N, (1, 1))

    @pl.when(i == pl.num_programs(0) - 1)
    def _():
        out_ref[...] += jnp.reshape(blin_ref[...], (1, 1))


def _final(a0, a1, scl, bias, wlin, blin):
    return pl.pallas_call(
        _final_body,
        grid=(5,),
        in_specs=_node_specs + [
            pl.BlockSpec((H, 1), lambda i: (0, 0)),
            pl.BlockSpec((1,), lambda i: (0,)),
        ],
        out_specs=pl.BlockSpec((1, 1), lambda i: (0, 0)),
        out_shape=jax.ShapeDtypeStruct((1, 1), f32),
    )(a0, a1, scl, bias, wlin, blin)


_sc_params = pltpu.CompilerParams(
    needs_layout_passes=False, use_tc_tiling_on_sc=False)


def _make_layer_sc():
    def body(src2_hbm, dst2_hbm, xl_hbm, xr_hbm, el_hbm, att_hbm,
             zn_hbm, znh_hbm, *rest):
        (ssum0_hbm, ssum1_hbm, cnt0_hbm, cnt1_hbm,
         acc0_hbm, acc1_hbm, den0_hbm, den1_hbm,
         srcv2, dstv2,
         xlv0, xlv1, xlv2, xlv3, xrv0, xrv1, xrv2, xrv3,
         ev0, ev1, ev2, ev3,
         avb, exb, shift_v, t0v, c0v, onesv, attv,
         ssum_sh, cnt_sh, acc_sh, den_sh,
         sgl0, sgl1, sgl2, sgl3, sgr0, sgr1, sgr2, sgr3,
         se0, se1, se2, se3, sss, ssc,
         srs0, srs1, srs2, srs3, sds) = rest
        xlv = [xlv0, xlv1, xlv2, xlv3]
        xrv = [xrv0, xrv1, xrv2, xrv3]
        ev = [ev0, ev1, ev2, ev3]
        rowv = xrv
        sgl = [sgl0, sgl1, sgl2, sgl3]
        sgr = [sgr0, sgr1, sgr2, sgr3]
        se = [se0, se1, se2, se3]
        srs = [srs0, srs1, srs2, srs3]

        c = lax.axis_index("c")
        s = lax.axis_index("s")
        wid = s * NC + c
        base0 = wid * EPW

        pltpu.sync_copy(att_hbm, attv)
        pltpu.sync_copy(src2_hbm.at[wid], srcv2)
        pltpu.sync_copy(dst2_hbm.at[wid], dstv2)

        ones16 = jnp.full((16,), 1.0, f32)

        def ofill(b, carry):
            onesv[pl.ds(16 * b, 16)] = ones16
            return carry

        lax.fori_loop(0, NG, ofill, 0)

        @pl.when(s == 0)
        def _():
            pltpu.sync_copy(zn_hbm, ssum_sh)
            pltpu.sync_copy(znh_hbm, acc_sh)
            pltpu.sync_copy(zn_hbm, den_sh)
            pltpu.sync_copy(zn_hbm, cnt_sh)

        plsc.subcore_barrier()

        attvec = attv[...]
        attks = [attvec[k] for k in range(H)]
        iota16 = lax.iota(jnp.int32, 16)
        kvecs = [jnp.full((16,), k, jnp.int32) for k in range(H)]

        def issue_g1(j, slot):
            pltpu.async_copy(xl_hbm.at[srcv2.at[j]], xlv[slot], sgl[slot])
            pltpu.async_copy(xr_hbm.at[dstv2.at[j]], xrv[slot], sgr[slot])
            pltpu.async_copy(el_hbm.at[pl.ds(base0 + j * C, C)],
                             ev[slot], se[slot])

        def process1(j, slot):
            pltpu.make_async_copy(xl_hbm.at[srcv2.at[j]], xlv[slot], sgl[slot]).wait()
            pltpu.make_async_copy(xr_hbm.at[dstv2.at[j]], xrv[slot], sgr[slot]).wait()
            pltpu.make_async_copy(el_hbm.at[pl.ds(base0 + j * C, C)],
                                  ev[slot], se[slot]).wait()
            def g1(b, carry):
                ivec = iota16 + b * 16
                acc = jnp.zeros((16,), f32)
                for k in range(H):
                    z = (plsc.load_gather(xlv[slot], [ivec, kvecs[k]])
                         + plsc.load_gather(xrv[slot], [ivec, kvecs[k]])
                         + plsc.load_gather(ev[slot], [ivec, kvecs[k]]))
                    m = jnp.maximum(z, 0.2 * z)
                    acc = acc + m * attks[k]
                avb[pl.ds(j * C + b * 16, 16)] = acc
                return carry

            lax.fori_loop(0, NG, g1, 0)
            pltpu.async_copy(avb.at[pl.ds(j * C, C)], ssum_sh.at[dstv2.at[j]],
                             sss, add=True)
            pltpu.async_copy(onesv, cnt_sh.at[dstv2.at[j]], ssc, add=True)

        issue_g1(0, 0)
        issue_g1(1, 1)
        issue_g1(2, 2)

        def quad1(t, carry):
            for q in range(4):
                j = 4 * t + q
                jn = j + 3

                @pl.when(jn < NCH)
                def _(jn=jn, q=q):
                    issue_g1(jn, (q + 3) % 4)

                process1(j, q)
            return carry

        lax.fori_loop(0, NCH // 4, quad1, 0)
        process1(NCH - 1, 0)

        def drain1(i, carry):
            pltpu.make_async_copy(avb.at[pl.ds(0, C)], ssum_sh.at[dstv2.at[0]],
                                  sss).wait()
            pltpu.make_async_copy(onesv, cnt_sh.at[dstv2.at[0]], ssc).wait()
            return carry

        lax.fori_loop(0, NCH, drain1, 0)

        plsc.subcore_barrier()

        pltpu.sync_copy(ssum_sh, t0v)
        pltpu.sync_copy(cnt_sh, c0v)

        def sbody(i, carry):
            sl = pl.ds(i * 16, 16)
            shift_v[sl] = t0v[sl] / jnp.maximum(c0v[sl], 1.0)
            return carry

        lax.fori_loop(0, N // 16, sbody, 0)

        @pl.when((s == 0) & (c == 0))
        def _():
            pltpu.sync_copy(ssum_sh, ssum0_hbm)
            pltpu.sync_copy(cnt_sh, cnt0_hbm)

        @pl.when((s == 0) & (c == 1))
        def _():
            pltpu.sync_copy(ssum_sh, ssum1_hbm)
            pltpu.sync_copy(cnt_sh, cnt1_hbm)

        def issue_g2(j, slot):
            pltpu.async_copy(xl_hbm.at[srcv2.at[j]], xlv[slot], sgl[slot])

        zero16 = jnp.zeros((16,), f32)
        for slot in range(4):
            def zfill(i, carry, _slot=slot):
                rowv[_slot][i, :] = zero16
                return carry

            lax.fori_loop(0, C, zfill, 0)
            pltpu.async_copy(rowv[slot], acc_sh.at[dstv2.at[0]], srs[slot],
                             add=True)

        issue_g2(0, 0)
        issue_g2(1, 1)
        issue_g2(2, 2)

        def process2(j, slot):
            pltpu.make_async_copy(xl_hbm.at[srcv2.at[j]], xlv[slot], sgl[slot]).wait()
            pltpu.make_async_copy(rowv[slot], acc_sh.at[dstv2.at[0]],
                                  srs[slot]).wait()
            def g2(b, carry):
                sl = pl.ds(j * C + b * 16, 16)
                dvec = dstv2[j, pl.ds(b * 16, 16)]
                svec = plsc.load_gather(shift_v, [dvec])
                exvec = jnp.exp(avb[sl] - svec)
                exb[sl] = exvec
                for t in range(16):
                    i = b * 16 + t
                    rowv[slot][i, :] = xlv[slot][i, :] * exvec[t]
                return carry

            lax.fori_loop(0, NG, g2, 0)
            pltpu.async_copy(exb.at[pl.ds(j * C, C)], den_sh.at[dstv2.at[j]],
                             sds, add=True)
            pltpu.async_copy(rowv[slot], acc_sh.at[dstv2.at[j]], srs[slot],
                             add=True)

        def quad2(t, carry):
            for q in range(4):
                j = 4 * t + q
                jn = j + 3

                @pl.when(jn < NCH)
                def _(jn=jn, q=q):
                    issue_g2(jn, (q + 3) % 4)

                process2(j, q)
            return carry

        lax.fori_loop(0, NCH // 4, quad2, 0)
        process2(NCH - 1, 0)

        def drain2(i, carry):
            pltpu.make_async_copy(exb.at[pl.ds(0, C)], den_sh.at[dstv2.at[0]],
                                  sds).wait()
            return carry

        lax.fori_loop(0, NCH, drain2, 0)
        for slot in range(4):
            pltpu.make_async_copy(rowv[slot], acc_sh.at[dstv2.at[0]],
                                  srs[slot]).wait()

        plsc.subcore_barrier()

        @pl.when((s == 0) & (c == 0))
        def _():
            pltpu.sync_copy(acc_sh, acc0_hbm)
            pltpu.sync_copy(den_sh, den0_hbm)

        @pl.when((s == 0) & (c == 1))
        def _():
            pltpu.sync_copy(acc_sh, acc1_hbm)
            pltpu.sync_copy(den_sh, den1_hbm)

    out_type = [jax.ShapeDtypeStruct((N,), f32),
                jax.ShapeDtypeStruct((N,), f32),
                jax.ShapeDtypeStruct((N,), f32),
                jax.ShapeDtypeStruct((N,), f32),
                jax.ShapeDtypeStruct((N, H), f32),
                jax.ShapeDtypeStruct((N, H), f32),
                jax.ShapeDtypeStruct((N,), f32),
                jax.ShapeDtypeStruct((N,), f32)]
    scratch = [
        pltpu.VMEM((NCH, C), jnp.int32),
        pltpu.VMEM((NCH, C), jnp.int32),
        pltpu.VMEM((C, H), f32),
        pltpu.VMEM((C, H), f32),
        pltpu.VMEM((C, H), f32),
        pltpu.VMEM((C, H), f32),
        pltpu.VMEM((C, H), f32),
        pltpu.VMEM((C, H), f32),
        pltpu.VMEM((C, H), f32),
        pltpu.VMEM((C, H), f32),
        pltpu.VMEM((C, H), f32),
        pltpu.VMEM((C, H), f32),
        pltpu.VMEM((C, H), f32),
        pltpu.VMEM((C, H), f32),
        pltpu.VMEM((EPW,), f32),
        pltpu.VMEM((EPW,), f32),
        pltpu.VMEM((N,), f32),
        pltpu.VMEM((N,), f32),
        pltpu.VMEM((N,), f32),
        pltpu.VMEM((C,), f32),
        pltpu.VMEM((16,), f32),
        pltpu.VMEM_SHARED((N,), f32),
        pltpu.VMEM_SHARED((N,), f32),
        pltpu.VMEM_SHARED((N, H), f32),
        pltpu.VMEM_SHARED((N,), f32),
    ]
    scratch += [pltpu.SemaphoreType.DMA] * 19

    return pl.kernel(body, out_type=out_type, mesh=_mesh,
                     scratch_types=scratch, compiler_params=_sc_params)


_layer_sc = _make_layer_sc()


def kernel(x, edge_index, edge_attr, params, Wlin, blin):
    src = edge_index[0].astype(jnp.int32)
    dst = edge_index[1].astype(jnp.int32)
    src2 = src.reshape(NW, NCH, C)
    dst2 = dst.reshape(NW, NCH, C)

    we3 = jnp.stack([p[4] for p in params])
    els = _edge_emb(edge_attr, we3)

    zn = jnp.zeros((N,), f32)
    znh = jnp.zeros((N, H), f32)

    wl, bl, wr, br, _, att, bias = params[0]
    xl, xr = _proj(x, wl, bl, wr, br)

    out = None
    for l in range(3):
        s0, s1, c0, c1, a0, a1, d0, d1 = _layer_sc(
            src2, dst2, xl, xr, els[l], att, zn, znh)
        scl = jnp.stack([s0, c0, d0, s1, c1, d1], axis=1)
        args = (a0, a1, scl, bias)
        if l < 2:
            nwl, nbl, nwr, nbr, _, natt, nbias = params[l + 1]
            xl, xr = _combine_proj(*args, nwl, nbl, nwr, nbr)
            att = natt
            bias = nbias
        else:
            out = _final(*args, Wlin, blin)
    return out
